# baseline (speedup 1.0000x reference)
import functools

import jax
import jax.numpy as jnp
from jax import lax
from jax.experimental import pallas as pl
from jax.experimental.pallas import tpu as pltpu
from jax.experimental.pallas import tpu_sc as plsc

D = 128
M = 100000
K = 64
L = 4
H = 8
DH = 16
FF = 512

MV_GRID = 32
MV_BLOCK = 3136
M_PAD = MV_GRID * MV_BLOCK
ROWS = M_PAD // 128

NEG = float("-inf")

NW = 32
CHUNK = M_PAD // NW
CVREGS = CHUNK // 16
NCAND = NW * K
MVREGS = NCAND // 16


def _sims_body(mem_ref, ctx_ref, out_ref):
    i = pl.program_id(0)
    s = jnp.dot(mem_ref[...], ctx_ref[...], preferred_element_type=jnp.float32)
    row = jax.lax.broadcasted_iota(jnp.int32, (MV_BLOCK, 1), 0) + i * MV_BLOCK
    out_ref[...] = jnp.where(row < M, s, NEG)


def _compute_sims(memory_tensor, ctx_col):
    return pl.pallas_call(
        _sims_body,
        grid=(MV_GRID,),
        in_specs=[
            pl.BlockSpec((MV_BLOCK, D), lambda i: (i, 0)),
            pl.BlockSpec((D, 1), lambda i: (0, 0)),
        ],
        out_specs=pl.BlockSpec((MV_BLOCK, 1), lambda i: (i, 0)),
        out_shape=jax.ShapeDtypeStruct((M_PAD, 1), jnp.float32),
    )(memory_tensor, ctx_col)


_SIGN_INT = -(2 ** 31)


def _f32_key(v):
    u = plsc.bitcast(v, jnp.int32)
    return jnp.where(u < 0, jnp.int32(-1) - (u + jnp.int32(_SIGN_INT)), u)


def _select_threshold(keys_ref, gidx_ref, nvregs, hist_v, tot_v, ck_v, cg_v, k):
    sign = jnp.int32(_SIGN_INT)
    lane = lax.iota(jnp.int32, 16)
    ones = jnp.ones((16,), jnp.int32)
    zeros = jnp.zeros((16,), jnp.int32)

    def zh(j, _):
        for u in range(8):
            hist_v[pl.ds(j * 128 + u * 16, 16)] = zeros
        return 0

    lax.fori_loop(0, 32, zh, 0)

    def hpass(j, _):
        for u in range(4):
            kv = keys_ref[pl.ds(j * 64 + u * 16, 16)]
            b = lax.shift_right_logical(kv + sign, jnp.int32(24))
            plsc.addupdate_scatter(hist_v, [lane * 256 + b], ones)
        return 0

    lax.fori_loop(0, nvregs // 4, hpass, 0)

    for g in range(16):
        acc = zeros
        for l in range(16):
            acc = acc + hist_v[pl.ds(l * 256 + g * 16, 16)]
        tot_v[pl.ds(g * 16, 16)] = acc

    run = jnp.int32(0)
    b_star = jnp.int32(0)
    found = jnp.int32(0)
    for g in range(15, -1, -1):
        tv = tot_v[pl.ds(g * 16, 16)]
        rv = lax.rev(tv, (0,))
        cs = plsc.cumsum(rv)
        m = (run + cs) >= k
        mi = jnp.where(m, 1, 0).astype(jnp.int32)
        any_m = jnp.sum(mi)
        mf = m & (plsc.cumsum(mi) == 1)
        i_star = jnp.sum(jnp.where(mf, lane, 0))
        b_g = g * 16 + 15 - i_star
        hit = (found == 0) & (any_m > 0)
        b_star = jnp.where(hit, b_g, b_star)
        found = jnp.where(hit, 1, found)
        run = run + jnp.sum(tv)

    ts8 = b_star * jnp.int32(1 << 24) + sign

    def cpass(j, off):
        for u in range(4):
            kv = keys_ref[pl.ds(j * 64 + u * 16, 16)]
            gv = gidx_ref[pl.ds(j * 64 + u * 16, 16)]
            m = kv >= ts8
            mi = jnp.where(m, 1, 0).astype(jnp.int32)
            pos = off + plsc.cumsum(mi) - mi
            plsc.store_scatter(ck_v, [pos], kv, mask=m)
            plsc.store_scatter(cg_v, [pos], gv, mask=m)
            off = off + jnp.sum(mi)
        return off

    nc = lax.fori_loop(0, nvregs // 4, cpass, jnp.int32(0))

    pad = jnp.broadcast_to(sign, (16,))
    for u in range(4):
        ck_v[pl.ds(nc + u * 16, 16)] = pad
    nv4 = lax.shift_right_logical(nc + 63, jnp.int32(6))

    def one_bit(_, carry):
        p, bit = carry
        t = p + bit
        ts = t + sign

        def cnt(j, acc):
            for u in range(4):
                kv = ck_v[pl.ds(j * 64 + u * 16, 16)]
                acc = acc + jnp.where(kv >= ts, 1, 0).astype(jnp.int32)
            return acc

        c = jnp.sum(lax.fori_loop(0, nv4, cnt, zeros))
        p = jnp.where(c >= k, t, p)
        return p, lax.shift_right_logical(bit, jnp.int32(1))

    p0 = b_star * jnp.int32(1 << 24)
    p, _ = lax.fori_loop(0, 24, one_bit, (p0, jnp.int32(1 << 23)))
    return p + sign, nc


def _sc_local_body(sims_hbm, ckeys_hbm, cgidx_hbm,
                   chunk_v, keys_v, gidx_v, hist_v, tot_v,
                   ck_v, cg_v, okey_v, ogid_v):
    wid = lax.axis_index("s") * 2 + lax.axis_index("c")
    base = wid * CHUNK
    pltpu.sync_copy(sims_hbm.at[pl.ds(base, CHUNK)], chunk_v)
    lane = lax.iota(jnp.int32, 16)

    def xform(j, _):
        for u in range(4):
            o = j * 64 + u * 16
            keys_v[pl.ds(o, 16)] = _f32_key(chunk_v[pl.ds(o, 16)])
            gidx_v[pl.ds(o, 16)] = base + o + lane
        return 0

    lax.fori_loop(0, CVREGS // 4, xform, 0)
    t_loc, nc = _select_threshold(keys_v, gidx_v, CVREGS, hist_v, tot_v,
                                  ck_v, cg_v, K)
    nv4 = lax.shift_right_logical(nc + 63, jnp.int32(6))

    def extract(j, off):
        for u in range(4):
            o = j * 64 + u * 16
            kv = ck_v[pl.ds(o, 16)]
            gv = cg_v[pl.ds(o, 16)]
            m = kv >= t_loc
            mi = jnp.where(m, 1, 0).astype(jnp.int32)
            pos = off + plsc.cumsum(mi) - mi
            plsc.store_scatter(okey_v, [pos], kv, mask=m)
            plsc.store_scatter(ogid_v, [pos], gv, mask=m)
            off = off + jnp.sum(mi)
        return off

    lax.fori_loop(0, nv4, extract, jnp.int32(0))
    pltpu.sync_copy(okey_v.at[pl.ds(0, K)], ckeys_hbm.at[pl.ds(wid * K, K)])
    pltpu.sync_copy(ogid_v.at[pl.ds(0, K)], cgidx_hbm.at[pl.ds(wid * K, K)])


def _sc_merge_body(ckeys_hbm, cgidx_hbm, mem_hbm, out_hbm,
                   k2_v, g2_v, hist_v, tot_v, ck_v, cg_v,
                   skey_v, sgid_v, sidx_v, rows_v, sem):
    wid = lax.axis_index("s") * 2 + lax.axis_index("c")

    @pl.when(wid == 0)
    def _():
        pltpu.sync_copy(ckeys_hbm, k2_v)
        pltpu.sync_copy(cgidx_hbm, g2_v)
        t_g, nc = _select_threshold(k2_v, g2_v, MVREGS, hist_v, tot_v,
                                    ck_v, cg_v, K)
        nv4 = lax.shift_right_logical(nc + 63, jnp.int32(6))

        def cnt_gt(j, acc):
            for u in range(4):
                kv = ck_v[pl.ds(j * 64 + u * 16, 16)]
                acc = acc + jnp.where(kv > t_g, 1, 0).astype(jnp.int32)
            return acc

        n_gt = jnp.sum(lax.fori_loop(0, nv4, cnt_gt, jnp.zeros((16,), jnp.int32)))
        r = K - n_gt

        def extract(j, carry):
            off, eq_seen = carry
            for u in range(4):
                o = j * 64 + u * 16
                kv = ck_v[pl.ds(o, 16)]
                gv = cg_v[pl.ds(o, 16)]
                m_gt = kv > t_g
                m_eq = kv == t_g
                ei = jnp.where(m_eq, 1, 0).astype(jnp.int32)
                eq_rank = eq_seen + plsc.cumsum(ei) - ei
                m = m_gt | (m_eq & (eq_rank < r))
                mi = jnp.where(m, 1, 0).astype(jnp.int32)
                pos = off + plsc.cumsum(mi) - mi
                plsc.store_scatter(skey_v, [pos], kv, mask=m)
                plsc.store_scatter(sgid_v, [pos], gv, mask=m)
                off = off + jnp.sum(mi)
                eq_seen = eq_seen + jnp.sum(ei)
            return off, eq_seen

        lax.fori_loop(0, nv4, extract, (jnp.int32(0), jnp.int32(0)))

        kvs = [skey_v[pl.ds(v * 16, 16)] for v in range(4)]
        gvs = [sgid_v[pl.ds(v * 16, 16)] for v in range(4)]

        def rank_step(j, ranks):
            jv = jnp.broadcast_to(j, (16,))
            kj = plsc.load_gather(skey_v, [jv])
            gj = plsc.load_gather(sgid_v, [jv])
            out = []
            for v in range(4):
                pre = (kj > kvs[v]) | ((kj == kvs[v]) & (gj < gvs[v]))
                out.append(ranks[v] + jnp.where(pre, 1, 0).astype(jnp.int32))
            return tuple(out)

        ranks = lax.fori_loop(0, K, rank_step,
                              tuple(jnp.zeros((16,), jnp.int32) for _ in range(4)))
        for v in range(4):
            plsc.store_scatter(sidx_v, [ranks[v]], gvs[v])

        pltpu.async_copy(mem_hbm.at[sidx_v], rows_v, sem).wait()
        pltpu.sync_copy(rows_v, out_hbm)


def _sc_topk_gather(sims_flat, memory_tensor):
    mesh = plsc.VectorSubcoreMesh(core_axis_name="c", subcore_axis_name="s")
    ckeys, cgidx = pl.kernel(
        _sc_local_body,
        out_type=(jax.ShapeDtypeStruct((NCAND,), jnp.int32),
                  jax.ShapeDtypeStruct((NCAND,), jnp.int32)),
        mesh=mesh,
        compiler_params=pltpu.CompilerParams(needs_layout_passes=False),
        scratch_types=[
            pltpu.VMEM((CHUNK,), jnp.float32),
            pltpu.VMEM((CHUNK,), jnp.int32),
            pltpu.VMEM((CHUNK,), jnp.int32),
            pltpu.VMEM((4096,), jnp.int32),
            pltpu.VMEM((256,), jnp.int32),
            pltpu.VMEM((CHUNK + 64,), jnp.int32),
            pltpu.VMEM((CHUNK + 64,), jnp.int32),
            pltpu.VMEM((CHUNK + 64,), jnp.int32),
            pltpu.VMEM((CHUNK + 64,), jnp.int32),
        ],
    )(sims_flat)
    return pl.kernel(
        _sc_merge_body,
        out_type=jax.ShapeDtypeStruct((K, D), jnp.float32),
        mesh=mesh,
        compiler_params=pltpu.CompilerParams(needs_layout_passes=False),
        scratch_types=[
            pltpu.VMEM((NCAND,), jnp.int32),
            pltpu.VMEM((NCAND,), jnp.int32),
            pltpu.VMEM((4096,), jnp.int32),
            pltpu.VMEM((256,), jnp.int32),
            pltpu.VMEM((NCAND + 64,), jnp.int32),
            pltpu.VMEM((NCAND + 64,), jnp.int32),
            pltpu.VMEM((K,), jnp.int32),
            pltpu.VMEM((K,), jnp.int32),
            pltpu.VMEM((K,), jnp.int32),
            pltpu.VMEM((K, D), jnp.float32),
            pltpu.SemaphoreType.DMA,
        ],
    )(ckeys, cgidx, memory_tensor)


def _topk_body(sims_ref, idx_ref):
    sims = sims_ref[...]
    r = jax.lax.broadcasted_iota(jnp.int32, (ROWS, 128), 0)
    c = jax.lax.broadcasted_iota(jnp.int32, (ROWS, 128), 1)
    flat = r * 128 + c

    def body(k, s):
        m = jnp.max(s)
        i = jnp.min(jnp.where(s == m, flat, jnp.int32(2**30)))
        idx_ref[k] = i
        return jnp.where(flat == i, NEG, s)

    jax.lax.fori_loop(0, K, body, sims)


def _topk_idx(sims2d):
    return pl.pallas_call(
        _topk_body,
        out_specs=pl.BlockSpec(memory_space=pltpu.SMEM),
        out_shape=jax.ShapeDtypeStruct((K,), jnp.int32),
    )(sims2d)


def _gather_body(idx_ref, mem_ref, out_ref):
    out_ref[...] = mem_ref[...]


def _gather_rows(idx, memory_tensor):
    grid_spec = pltpu.PrefetchScalarGridSpec(
        num_scalar_prefetch=1,
        grid=(K,),
        in_specs=[pl.BlockSpec((1, 1, D), lambda i, idx_ref: (idx_ref[i], 0, 0))],
        out_specs=pl.BlockSpec((1, 1, D), lambda i, idx_ref: (i, 0, 0)),
    )
    out = pl.pallas_call(
        _gather_body,
        grid_spec=grid_spec,
        out_shape=jax.ShapeDtypeStruct((K, 1, D), jnp.float32),
    )(idx, memory_tensor.reshape(M, 1, D))
    return out.reshape(K, D)


def _layer_norm_in(x, w, b):
    m = jnp.mean(x, axis=-1, keepdims=True)
    d = x - m
    v = jnp.mean(d * d, axis=-1, keepdims=True)
    return d * (1.0 / jnp.sqrt(v + 1e-5)) * w + b


def _tfm_body(x0_ref, wi_ref, bi_ref, wo_ref, bo_ref, w1_ref, b1_ref,
              w2_ref, b2_ref, l1w_ref, l1b_ref, l2w_ref, l2b_ref, out_ref):
    x = x0_ref[...]
    inv_sqrt = 1.0 / (DH ** 0.5)
    for l in range(L):
        qkv = jnp.dot(x, wi_ref[l], preferred_element_type=jnp.float32) + bi_ref[l]
        heads = []
        for h in range(H):
            q = qkv[:, h * DH:(h + 1) * DH]
            k = qkv[:, D + h * DH:D + (h + 1) * DH]
            v = qkv[:, 2 * D + h * DH:2 * D + (h + 1) * DH]
            scores = jax.lax.dot_general(
                q, k, (((1,), (1,)), ((), ())),
                preferred_element_type=jnp.float32) * inv_sqrt
            mx = jnp.max(scores, axis=-1, keepdims=True)
            e = jnp.exp(scores - mx)
            attn = e / jnp.sum(e, axis=-1, keepdims=True)
            heads.append(jnp.dot(attn, v, preferred_element_type=jnp.float32))
        a = jnp.concatenate(heads, axis=1)
        a = jnp.dot(a, wo_ref[l], preferred_element_type=jnp.float32) + bo_ref[l]
        x = _layer_norm_in(x + a, l1w_ref[l], l1b_ref[l])
        ff = jnp.maximum(jnp.dot(x, w1_ref[l], preferred_element_type=jnp.float32) + b1_ref[l], 0.0)
        ff = jnp.dot(ff, w2_ref[l], preferred_element_type=jnp.float32) + b2_ref[l]
        x = _layer_norm_in(x + ff, l2w_ref[l], l2b_ref[l])
    out_ref[...] = x


def _transformer(x0, wi_t, bi, wo_t, bo, w1_t, b1, w2_t, b2, l1w, l1b, l2w, l2b):
    return pl.pallas_call(
        _tfm_body,
        out_shape=jax.ShapeDtypeStruct((K, D), jnp.float32),
    )(x0, wi_t, bi, wo_t, bo, w1_t, b1, w2_t, b2, l1w, l1b, l2w, l2b)


def kernel(current_context, memory_tensor, W_in, b_in, W_out, b_out,
           W1, b1, W2, b2, ln1_w, ln1_b, ln2_w, ln2_b, top_k):
    ctx_col = current_context.reshape(D, 1)
    sims = _compute_sims(memory_tensor, ctx_col)
    x0 = _sc_topk_gather(sims.reshape(M_PAD), memory_tensor)

    wi_t = jnp.transpose(W_in, (0, 2, 1))
    wo_t = jnp.transpose(W_out, (0, 2, 1))
    w1_t = jnp.transpose(W1, (0, 2, 1))
    w2_t = jnp.transpose(W2, (0, 2, 1))
    bi = b_in.reshape(L, 1, 3 * D)
    bo = b_out.reshape(L, 1, D)
    b1r = b1.reshape(L, 1, FF)
    b2r = b2.reshape(L, 1, D)
    l1w = ln1_w.reshape(L, 1, D)
    l1b = ln1_b.reshape(L, 1, D)
    l2w = ln2_w.reshape(L, 1, D)
    l2b = ln2_b.reshape(L, 1, D)
    return _transformer(x0, wi_t, bi, wo_t, bo, w1_t, b1r, w2_t, b2r,
                        l1w, l1b, l2w, l2b)


# speedup vs baseline: 1.0936x; 1.0936x over previous
import functools

import jax
import jax.numpy as jnp
from jax import lax
from jax.experimental import pallas as pl
from jax.experimental.pallas import tpu as pltpu
from jax.experimental.pallas import tpu_sc as plsc

D = 128
M = 100000
K = 64
L = 4
H = 8
DH = 16
FF = 512

MV_GRID = 8
MV_BLOCK = 12544
M_PAD = MV_GRID * MV_BLOCK
ROWS = M_PAD // 128

NEG = float("-inf")

NW = 32
CHUNK = M_PAD // NW
CVREGS = CHUNK // 16
NCAND = NW * K
MVREGS = NCAND // 16


def _sims_body(mem_ref, ctx_ref, out_ref):
    i = pl.program_id(0)
    s = jnp.dot(mem_ref[...], ctx_ref[...], preferred_element_type=jnp.float32)
    row = jax.lax.broadcasted_iota(jnp.int32, (MV_BLOCK, 1), 0) + i * MV_BLOCK
    out_ref[...] = jnp.where(row < M, s, NEG)


def _compute_sims(memory_tensor, ctx_col):
    return pl.pallas_call(
        _sims_body,
        grid=(MV_GRID,),
        in_specs=[
            pl.BlockSpec((MV_BLOCK, D), lambda i: (i, 0)),
            pl.BlockSpec((D, 1), lambda i: (0, 0)),
        ],
        out_specs=pl.BlockSpec((MV_BLOCK, 1), lambda i: (i, 0)),
        out_shape=jax.ShapeDtypeStruct((M_PAD, 1), jnp.float32),
    )(memory_tensor, ctx_col)


_SIGN_INT = -(2 ** 31)


def _f32_key(v):
    u = plsc.bitcast(v, jnp.int32)
    return jnp.where(u < 0, jnp.int32(-1) - (u + jnp.int32(_SIGN_INT)), u)


def _select_threshold(keys_ref, gidx_ref, nvregs, hist_v, tot_v, ck_v, cg_v, k):
    sign = jnp.int32(_SIGN_INT)
    lane = lax.iota(jnp.int32, 16)
    ones = jnp.ones((16,), jnp.int32)
    zeros = jnp.zeros((16,), jnp.int32)

    def zh(j, _):
        for u in range(8):
            hist_v[pl.ds(j * 128 + u * 16, 16)] = zeros
        return 0

    lax.fori_loop(0, 32, zh, 0)

    def hpass(j, _):
        for u in range(4):
            kv = keys_ref[pl.ds(j * 64 + u * 16, 16)]
            b = lax.shift_right_logical(kv + sign, jnp.int32(24))
            plsc.addupdate_scatter(hist_v, [lane * 256 + b], ones)
        return 0

    lax.fori_loop(0, nvregs // 4, hpass, 0)

    for g in range(16):
        acc = zeros
        for l in range(16):
            acc = acc + hist_v[pl.ds(l * 256 + g * 16, 16)]
        tot_v[pl.ds(g * 16, 16)] = acc

    run = jnp.int32(0)
    b_star = jnp.int32(0)
    found = jnp.int32(0)
    for g in range(15, -1, -1):
        tv = tot_v[pl.ds(g * 16, 16)]
        rv = lax.rev(tv, (0,))
        cs = plsc.cumsum(rv)
        m = (run + cs) >= k
        mi = jnp.where(m, 1, 0).astype(jnp.int32)
        any_m = jnp.sum(mi)
        mf = m & (plsc.cumsum(mi) == 1)
        i_star = jnp.sum(jnp.where(mf, lane, 0))
        b_g = g * 16 + 15 - i_star
        hit = (found == 0) & (any_m > 0)
        b_star = jnp.where(hit, b_g, b_star)
        found = jnp.where(hit, 1, found)
        run = run + jnp.sum(tv)

    ts8 = b_star * jnp.int32(1 << 24) + sign

    def cpass(j, off):
        for u in range(4):
            kv = keys_ref[pl.ds(j * 64 + u * 16, 16)]
            gv = gidx_ref[pl.ds(j * 64 + u * 16, 16)]
            m = kv >= ts8
            mi = jnp.where(m, 1, 0).astype(jnp.int32)
            pos = off + plsc.cumsum(mi) - mi
            plsc.store_scatter(ck_v, [pos], kv, mask=m)
            plsc.store_scatter(cg_v, [pos], gv, mask=m)
            off = off + jnp.sum(mi)
        return off

    nc = lax.fori_loop(0, nvregs // 4, cpass, jnp.int32(0))

    pad = jnp.broadcast_to(sign, (16,))
    for u in range(4):
        ck_v[pl.ds(nc + u * 16, 16)] = pad
    nv4 = lax.shift_right_logical(nc + 63, jnp.int32(6))

    def one_bit(_, carry):
        p, bit = carry
        t = p + bit
        ts = t + sign

        def cnt(j, acc):
            for u in range(4):
                kv = ck_v[pl.ds(j * 64 + u * 16, 16)]
                acc = acc + jnp.where(kv >= ts, 1, 0).astype(jnp.int32)
            return acc

        c = jnp.sum(lax.fori_loop(0, nv4, cnt, zeros))
        p = jnp.where(c >= k, t, p)
        return p, lax.shift_right_logical(bit, jnp.int32(1))

    p0 = b_star * jnp.int32(1 << 24)
    p, _ = lax.fori_loop(0, 24, one_bit, (p0, jnp.int32(1 << 23)))
    return p + sign, nc


def _sc_local_body(sims_hbm, ckeys_hbm, cgidx_hbm,
                   chunk_v, keys_v, gidx_v, hist_v, tot_v,
                   ck_v, cg_v, okey_v, ogid_v):
    wid = lax.axis_index("s") * 2 + lax.axis_index("c")
    base = wid * CHUNK
    pltpu.sync_copy(sims_hbm.at[pl.ds(base, CHUNK)], chunk_v)
    lane = lax.iota(jnp.int32, 16)

    def xform(j, _):
        for u in range(4):
            o = j * 64 + u * 16
            keys_v[pl.ds(o, 16)] = _f32_key(chunk_v[pl.ds(o, 16)])
            gidx_v[pl.ds(o, 16)] = base + o + lane
        return 0

    lax.fori_loop(0, CVREGS // 4, xform, 0)
    t_loc, nc = _select_threshold(keys_v, gidx_v, CVREGS, hist_v, tot_v,
                                  ck_v, cg_v, K)
    nv4 = lax.shift_right_logical(nc + 63, jnp.int32(6))

    def extract(j, off):
        for u in range(4):
            o = j * 64 + u * 16
            kv = ck_v[pl.ds(o, 16)]
            gv = cg_v[pl.ds(o, 16)]
            m = kv >= t_loc
            mi = jnp.where(m, 1, 0).astype(jnp.int32)
            pos = off + plsc.cumsum(mi) - mi
            plsc.store_scatter(okey_v, [pos], kv, mask=m)
            plsc.store_scatter(ogid_v, [pos], gv, mask=m)
            off = off + jnp.sum(mi)
        return off

    lax.fori_loop(0, nv4, extract, jnp.int32(0))
    pltpu.sync_copy(okey_v.at[pl.ds(0, K)], ckeys_hbm.at[pl.ds(wid * K, K)])
    pltpu.sync_copy(ogid_v.at[pl.ds(0, K)], cgidx_hbm.at[pl.ds(wid * K, K)])


def _sc_merge_body(ckeys_hbm, cgidx_hbm, mem_hbm, out_hbm,
                   k2_v, g2_v, hist_v, tot_v, ck_v, cg_v,
                   skey_v, sgid_v, sidx_v, rows_v, sem):
    wid = lax.axis_index("s") * 2 + lax.axis_index("c")

    @pl.when(wid == 0)
    def _():
        pltpu.sync_copy(ckeys_hbm, k2_v)
        pltpu.sync_copy(cgidx_hbm, g2_v)
        t_g, nc = _select_threshold(k2_v, g2_v, MVREGS, hist_v, tot_v,
                                    ck_v, cg_v, K)
        nv4 = lax.shift_right_logical(nc + 63, jnp.int32(6))

        def cnt_gt(j, acc):
            for u in range(4):
                kv = ck_v[pl.ds(j * 64 + u * 16, 16)]
                acc = acc + jnp.where(kv > t_g, 1, 0).astype(jnp.int32)
            return acc

        n_gt = jnp.sum(lax.fori_loop(0, nv4, cnt_gt, jnp.zeros((16,), jnp.int32)))
        r = K - n_gt

        def extract(j, carry):
            off, eq_seen = carry
            for u in range(4):
                o = j * 64 + u * 16
                kv = ck_v[pl.ds(o, 16)]
                gv = cg_v[pl.ds(o, 16)]
                m_gt = kv > t_g
                m_eq = kv == t_g
                ei = jnp.where(m_eq, 1, 0).astype(jnp.int32)
                eq_rank = eq_seen + plsc.cumsum(ei) - ei
                m = m_gt | (m_eq & (eq_rank < r))
                mi = jnp.where(m, 1, 0).astype(jnp.int32)
                pos = off + plsc.cumsum(mi) - mi
                plsc.store_scatter(skey_v, [pos], kv, mask=m)
                plsc.store_scatter(sgid_v, [pos], gv, mask=m)
                off = off + jnp.sum(mi)
                eq_seen = eq_seen + jnp.sum(ei)
            return off, eq_seen

        lax.fori_loop(0, nv4, extract, (jnp.int32(0), jnp.int32(0)))

        kvs = [skey_v[pl.ds(v * 16, 16)] for v in range(4)]
        gvs = [sgid_v[pl.ds(v * 16, 16)] for v in range(4)]

        def rank_step(j, ranks):
            jv = jnp.broadcast_to(j, (16,))
            kj = plsc.load_gather(skey_v, [jv])
            gj = plsc.load_gather(sgid_v, [jv])
            out = []
            for v in range(4):
                pre = (kj > kvs[v]) | ((kj == kvs[v]) & (gj < gvs[v]))
                out.append(ranks[v] + jnp.where(pre, 1, 0).astype(jnp.int32))
            return tuple(out)

        ranks = lax.fori_loop(0, K, rank_step,
                              tuple(jnp.zeros((16,), jnp.int32) for _ in range(4)))
        for v in range(4):
            plsc.store_scatter(sidx_v, [ranks[v]], gvs[v])

        pltpu.async_copy(mem_hbm.at[sidx_v], rows_v, sem).wait()
        pltpu.sync_copy(rows_v, out_hbm)


def _sc_topk_gather(sims_flat, memory_tensor):
    mesh = plsc.VectorSubcoreMesh(core_axis_name="c", subcore_axis_name="s")
    ckeys, cgidx = pl.kernel(
        _sc_local_body,
        out_type=(jax.ShapeDtypeStruct((NCAND,), jnp.int32),
                  jax.ShapeDtypeStruct((NCAND,), jnp.int32)),
        mesh=mesh,
        compiler_params=pltpu.CompilerParams(needs_layout_passes=False),
        scratch_types=[
            pltpu.VMEM((CHUNK,), jnp.float32),
            pltpu.VMEM((CHUNK,), jnp.int32),
            pltpu.VMEM((CHUNK,), jnp.int32),
            pltpu.VMEM((4096,), jnp.int32),
            pltpu.VMEM((256,), jnp.int32),
            pltpu.VMEM((CHUNK + 64,), jnp.int32),
            pltpu.VMEM((CHUNK + 64,), jnp.int32),
            pltpu.VMEM((CHUNK + 64,), jnp.int32),
            pltpu.VMEM((CHUNK + 64,), jnp.int32),
        ],
    )(sims_flat)
    return pl.kernel(
        _sc_merge_body,
        out_type=jax.ShapeDtypeStruct((K, D), jnp.float32),
        mesh=mesh,
        compiler_params=pltpu.CompilerParams(needs_layout_passes=False),
        scratch_types=[
            pltpu.VMEM((NCAND,), jnp.int32),
            pltpu.VMEM((NCAND,), jnp.int32),
            pltpu.VMEM((4096,), jnp.int32),
            pltpu.VMEM((256,), jnp.int32),
            pltpu.VMEM((NCAND + 64,), jnp.int32),
            pltpu.VMEM((NCAND + 64,), jnp.int32),
            pltpu.VMEM((K,), jnp.int32),
            pltpu.VMEM((K,), jnp.int32),
            pltpu.VMEM((K,), jnp.int32),
            pltpu.VMEM((K, D), jnp.float32),
            pltpu.SemaphoreType.DMA,
        ],
    )(ckeys, cgidx, memory_tensor)


def _topk_body(sims_ref, idx_ref):
    sims = sims_ref[...]
    r = jax.lax.broadcasted_iota(jnp.int32, (ROWS, 128), 0)
    c = jax.lax.broadcasted_iota(jnp.int32, (ROWS, 128), 1)
    flat = r * 128 + c

    def body(k, s):
        m = jnp.max(s)
        i = jnp.min(jnp.where(s == m, flat, jnp.int32(2**30)))
        idx_ref[k] = i
        return jnp.where(flat == i, NEG, s)

    jax.lax.fori_loop(0, K, body, sims)


def _topk_idx(sims2d):
    return pl.pallas_call(
        _topk_body,
        out_specs=pl.BlockSpec(memory_space=pltpu.SMEM),
        out_shape=jax.ShapeDtypeStruct((K,), jnp.int32),
    )(sims2d)


def _gather_body(idx_ref, mem_ref, out_ref):
    out_ref[...] = mem_ref[...]


def _gather_rows(idx, memory_tensor):
    grid_spec = pltpu.PrefetchScalarGridSpec(
        num_scalar_prefetch=1,
        grid=(K,),
        in_specs=[pl.BlockSpec((1, 1, D), lambda i, idx_ref: (idx_ref[i], 0, 0))],
        out_specs=pl.BlockSpec((1, 1, D), lambda i, idx_ref: (i, 0, 0)),
    )
    out = pl.pallas_call(
        _gather_body,
        grid_spec=grid_spec,
        out_shape=jax.ShapeDtypeStruct((K, 1, D), jnp.float32),
    )(idx, memory_tensor.reshape(M, 1, D))
    return out.reshape(K, D)


def _layer_norm_in(x, w, b):
    m = jnp.mean(x, axis=-1, keepdims=True)
    d = x - m
    v = jnp.mean(d * d, axis=-1, keepdims=True)
    return d * (1.0 / jnp.sqrt(v + 1e-5)) * w + b


def _tfm_body(x0_ref, wi_ref, bi_ref, wo_ref, bo_ref, w1_ref, b1_ref,
              w2_ref, b2_ref, l1w_ref, l1b_ref, l2w_ref, l2b_ref, out_ref):
    x = x0_ref[...]
    inv_sqrt = 1.0 / (DH ** 0.5)
    for l in range(L):
        qkv = jnp.dot(x, wi_ref[l], preferred_element_type=jnp.float32) + bi_ref[l]
        heads = []
        for h in range(H):
            q = qkv[:, h * DH:(h + 1) * DH]
            k = qkv[:, D + h * DH:D + (h + 1) * DH]
            v = qkv[:, 2 * D + h * DH:2 * D + (h + 1) * DH]
            scores = jax.lax.dot_general(
                q, k, (((1,), (1,)), ((), ())),
                preferred_element_type=jnp.float32) * inv_sqrt
            mx = jnp.max(scores, axis=-1, keepdims=True)
            e = jnp.exp(scores - mx)
            attn = e / jnp.sum(e, axis=-1, keepdims=True)
            heads.append(jnp.dot(attn, v, preferred_element_type=jnp.float32))
        a = jnp.concatenate(heads, axis=1)
        a = jnp.dot(a, wo_ref[l], preferred_element_type=jnp.float32) + bo_ref[l]
        x = _layer_norm_in(x + a, l1w_ref[l], l1b_ref[l])
        ff = jnp.maximum(jnp.dot(x, w1_ref[l], preferred_element_type=jnp.float32) + b1_ref[l], 0.0)
        ff = jnp.dot(ff, w2_ref[l], preferred_element_type=jnp.float32) + b2_ref[l]
        x = _layer_norm_in(x + ff, l2w_ref[l], l2b_ref[l])
    out_ref[...] = x


def _transformer(x0, wi_t, bi, wo_t, bo, w1_t, b1, w2_t, b2, l1w, l1b, l2w, l2b):
    return pl.pallas_call(
        _tfm_body,
        out_shape=jax.ShapeDtypeStruct((K, D), jnp.float32),
    )(x0, wi_t, bi, wo_t, bo, w1_t, b1, w2_t, b2, l1w, l1b, l2w, l2b)


def kernel(current_context, memory_tensor, W_in, b_in, W_out, b_out,
           W1, b1, W2, b2, ln1_w, ln1_b, ln2_w, ln2_b, top_k):
    ctx_col = current_context.reshape(D, 1)
    sims = _compute_sims(memory_tensor, ctx_col)
    x0 = _sc_topk_gather(sims.reshape(M_PAD), memory_tensor)

    wi_t = jnp.transpose(W_in, (0, 2, 1))
    wo_t = jnp.transpose(W_out, (0, 2, 1))
    w1_t = jnp.transpose(W1, (0, 2, 1))
    w2_t = jnp.transpose(W2, (0, 2, 1))
    bi = b_in.reshape(L, 1, 3 * D)
    bo = b_out.reshape(L, 1, D)
    b1r = b1.reshape(L, 1, FF)
    b2r = b2.reshape(L, 1, D)
    l1w = ln1_w.reshape(L, 1, D)
    l1b = ln1_b.reshape(L, 1, D)
    l2w = ln2_w.reshape(L, 1, D)
    l2b = ln2_b.reshape(L, 1, D)
    return _transformer(x0, wi_t, bi, wo_t, bo, w1_t, b1r, w2_t, b2r,
                        l1w, l1b, l2w, l2b)


# speedup vs baseline: 1.1107x; 1.0156x over previous
import functools

import jax
import jax.numpy as jnp
from jax import lax
from jax.experimental import pallas as pl
from jax.experimental.pallas import tpu as pltpu
from jax.experimental.pallas import tpu_sc as plsc

D = 128
M = 100000
K = 64
L = 4
H = 8
DH = 16
FF = 512

MV_GRID = 4
MV_BLOCK = 25088
M_PAD = MV_GRID * MV_BLOCK
ROWS = M_PAD // 128

NEG = float("-inf")

NW = 32
CHUNK = M_PAD // NW
CVREGS = CHUNK // 16
NCAND = NW * K
MVREGS = NCAND // 16


def _sims_body(mem_ref, ctx_ref, out_ref):
    i = pl.program_id(0)
    s = jnp.dot(mem_ref[...], ctx_ref[...], preferred_element_type=jnp.float32)
    row = jax.lax.broadcasted_iota(jnp.int32, (MV_BLOCK, 1), 0) + i * MV_BLOCK
    out_ref[...] = jnp.where(row < M, s, NEG)


def _compute_sims(memory_tensor, ctx_col):
    return pl.pallas_call(
        _sims_body,
        grid=(MV_GRID,),
        in_specs=[
            pl.BlockSpec((MV_BLOCK, D), lambda i: (i, 0)),
            pl.BlockSpec((D, 1), lambda i: (0, 0)),
        ],
        out_specs=pl.BlockSpec((MV_BLOCK, 1), lambda i: (i, 0)),
        out_shape=jax.ShapeDtypeStruct((M_PAD, 1), jnp.float32),
    )(memory_tensor, ctx_col)


_SIGN_INT = -(2 ** 31)


def _f32_key(v):
    u = plsc.bitcast(v, jnp.int32)
    return jnp.where(u < 0, jnp.int32(-1) - (u + jnp.int32(_SIGN_INT)), u)


def _select_threshold(keys_ref, gidx_ref, nvregs, hist_v, tot_v, ck_v, cg_v, k):
    sign = jnp.int32(_SIGN_INT)
    lane = lax.iota(jnp.int32, 16)
    ones = jnp.ones((16,), jnp.int32)
    zeros = jnp.zeros((16,), jnp.int32)

    def zh(j, _):
        for u in range(8):
            hist_v[pl.ds(j * 128 + u * 16, 16)] = zeros
        return 0

    lax.fori_loop(0, 32, zh, 0)

    def hpass(j, _):
        for u in range(4):
            kv = keys_ref[pl.ds(j * 64 + u * 16, 16)]
            b = lax.shift_right_logical(kv + sign, jnp.int32(24))
            plsc.addupdate_scatter(hist_v, [lane * 256 + b], ones)
        return 0

    lax.fori_loop(0, nvregs // 4, hpass, 0)

    for g in range(16):
        acc = zeros
        for l in range(16):
            acc = acc + hist_v[pl.ds(l * 256 + g * 16, 16)]
        tot_v[pl.ds(g * 16, 16)] = acc

    run = jnp.int32(0)
    b_star = jnp.int32(0)
    found = jnp.int32(0)
    for g in range(15, -1, -1):
        tv = tot_v[pl.ds(g * 16, 16)]
        rv = lax.rev(tv, (0,))
        cs = plsc.cumsum(rv)
        m = (run + cs) >= k
        mi = jnp.where(m, 1, 0).astype(jnp.int32)
        any_m = jnp.sum(mi)
        mf = m & (plsc.cumsum(mi) == 1)
        i_star = jnp.sum(jnp.where(mf, lane, 0))
        b_g = g * 16 + 15 - i_star
        hit = (found == 0) & (any_m > 0)
        b_star = jnp.where(hit, b_g, b_star)
        found = jnp.where(hit, 1, found)
        run = run + jnp.sum(tv)

    ts8 = b_star * jnp.int32(1 << 24) + sign

    def cpass(j, off):
        for u in range(4):
            kv = keys_ref[pl.ds(j * 64 + u * 16, 16)]
            gv = gidx_ref[pl.ds(j * 64 + u * 16, 16)]
            m = kv >= ts8
            mi = jnp.where(m, 1, 0).astype(jnp.int32)
            pos = off + plsc.cumsum(mi) - mi
            plsc.store_scatter(ck_v, [pos], kv, mask=m)
            plsc.store_scatter(cg_v, [pos], gv, mask=m)
            off = off + jnp.sum(mi)
        return off

    nc = lax.fori_loop(0, nvregs // 4, cpass, jnp.int32(0))

    pad = jnp.broadcast_to(sign, (16,))
    for u in range(4):
        ck_v[pl.ds(nc + u * 16, 16)] = pad
    nv4 = lax.shift_right_logical(nc + 63, jnp.int32(6))

    def one_bit(_, carry):
        p, bit = carry
        t = p + bit
        ts = t + sign

        def cnt(j, acc):
            for u in range(4):
                kv = ck_v[pl.ds(j * 64 + u * 16, 16)]
                acc = acc + jnp.where(kv >= ts, 1, 0).astype(jnp.int32)
            return acc

        c = jnp.sum(lax.fori_loop(0, nv4, cnt, zeros))
        p = jnp.where(c >= k, t, p)
        return p, lax.shift_right_logical(bit, jnp.int32(1))

    p0 = b_star * jnp.int32(1 << 24)
    p, _ = lax.fori_loop(0, 24, one_bit, (p0, jnp.int32(1 << 23)))
    return p + sign, nc


def _sc_local_body(sims_hbm, ckeys_hbm, cgidx_hbm,
                   chunk_v, keys_v, gidx_v, hist_v, tot_v,
                   ck_v, cg_v, okey_v, ogid_v):
    wid = lax.axis_index("s") * 2 + lax.axis_index("c")
    base = wid * CHUNK
    pltpu.sync_copy(sims_hbm.at[pl.ds(base, CHUNK)], chunk_v)
    lane = lax.iota(jnp.int32, 16)

    def xform(j, _):
        for u in range(4):
            o = j * 64 + u * 16
            keys_v[pl.ds(o, 16)] = _f32_key(chunk_v[pl.ds(o, 16)])
            gidx_v[pl.ds(o, 16)] = base + o + lane
        return 0

    lax.fori_loop(0, CVREGS // 4, xform, 0)
    t_loc, nc = _select_threshold(keys_v, gidx_v, CVREGS, hist_v, tot_v,
                                  ck_v, cg_v, K)
    nv4 = lax.shift_right_logical(nc + 63, jnp.int32(6))

    def extract(j, off):
        for u in range(4):
            o = j * 64 + u * 16
            kv = ck_v[pl.ds(o, 16)]
            gv = cg_v[pl.ds(o, 16)]
            m = kv >= t_loc
            mi = jnp.where(m, 1, 0).astype(jnp.int32)
            pos = off + plsc.cumsum(mi) - mi
            plsc.store_scatter(okey_v, [pos], kv, mask=m)
            plsc.store_scatter(ogid_v, [pos], gv, mask=m)
            off = off + jnp.sum(mi)
        return off

    lax.fori_loop(0, nv4, extract, jnp.int32(0))
    pltpu.sync_copy(okey_v.at[pl.ds(0, K)], ckeys_hbm.at[pl.ds(wid * K, K)])
    pltpu.sync_copy(ogid_v.at[pl.ds(0, K)], cgidx_hbm.at[pl.ds(wid * K, K)])


def _sc_merge_body(ckeys_hbm, cgidx_hbm, mem_hbm, out_hbm,
                   k2_v, g2_v, hist_v, tot_v, ck_v, cg_v,
                   skey_v, sgid_v, sidx_v, rows_v, sem):
    wid = lax.axis_index("s") * 2 + lax.axis_index("c")

    @pl.when(wid == 0)
    def _():
        pltpu.sync_copy(ckeys_hbm, k2_v)
        pltpu.sync_copy(cgidx_hbm, g2_v)
        t_g, nc = _select_threshold(k2_v, g2_v, MVREGS, hist_v, tot_v,
                                    ck_v, cg_v, K)
        nv4 = lax.shift_right_logical(nc + 63, jnp.int32(6))

        def cnt_gt(j, acc):
            for u in range(4):
                kv = ck_v[pl.ds(j * 64 + u * 16, 16)]
                acc = acc + jnp.where(kv > t_g, 1, 0).astype(jnp.int32)
            return acc

        n_gt = jnp.sum(lax.fori_loop(0, nv4, cnt_gt, jnp.zeros((16,), jnp.int32)))
        r = K - n_gt

        def extract(j, carry):
            off, eq_seen = carry
            for u in range(4):
                o = j * 64 + u * 16
                kv = ck_v[pl.ds(o, 16)]
                gv = cg_v[pl.ds(o, 16)]
                m_gt = kv > t_g
                m_eq = kv == t_g
                ei = jnp.where(m_eq, 1, 0).astype(jnp.int32)
                eq_rank = eq_seen + plsc.cumsum(ei) - ei
                m = m_gt | (m_eq & (eq_rank < r))
                mi = jnp.where(m, 1, 0).astype(jnp.int32)
                pos = off + plsc.cumsum(mi) - mi
                plsc.store_scatter(skey_v, [pos], kv, mask=m)
                plsc.store_scatter(sgid_v, [pos], gv, mask=m)
                off = off + jnp.sum(mi)
                eq_seen = eq_seen + jnp.sum(ei)
            return off, eq_seen

        lax.fori_loop(0, nv4, extract, (jnp.int32(0), jnp.int32(0)))

        kvs = [skey_v[pl.ds(v * 16, 16)] for v in range(4)]
        gvs = [sgid_v[pl.ds(v * 16, 16)] for v in range(4)]

        def rank_step(j, ranks):
            jv = jnp.broadcast_to(j, (16,))
            kj = plsc.load_gather(skey_v, [jv])
            gj = plsc.load_gather(sgid_v, [jv])
            out = []
            for v in range(4):
                pre = (kj > kvs[v]) | ((kj == kvs[v]) & (gj < gvs[v]))
                out.append(ranks[v] + jnp.where(pre, 1, 0).astype(jnp.int32))
            return tuple(out)

        ranks = lax.fori_loop(0, K, rank_step,
                              tuple(jnp.zeros((16,), jnp.int32) for _ in range(4)))
        for v in range(4):
            plsc.store_scatter(sidx_v, [ranks[v]], gvs[v])

        pltpu.async_copy(mem_hbm.at[sidx_v], rows_v, sem).wait()
        pltpu.sync_copy(rows_v, out_hbm)


def _sc_topk_gather(sims_flat, memory_tensor):
    mesh = plsc.VectorSubcoreMesh(core_axis_name="c", subcore_axis_name="s")
    ckeys, cgidx = pl.kernel(
        _sc_local_body,
        out_type=(jax.ShapeDtypeStruct((NCAND,), jnp.int32),
                  jax.ShapeDtypeStruct((NCAND,), jnp.int32)),
        mesh=mesh,
        compiler_params=pltpu.CompilerParams(needs_layout_passes=False),
        scratch_types=[
            pltpu.VMEM((CHUNK,), jnp.float32),
            pltpu.VMEM((CHUNK,), jnp.int32),
            pltpu.VMEM((CHUNK,), jnp.int32),
            pltpu.VMEM((4096,), jnp.int32),
            pltpu.VMEM((256,), jnp.int32),
            pltpu.VMEM((CHUNK + 64,), jnp.int32),
            pltpu.VMEM((CHUNK + 64,), jnp.int32),
            pltpu.VMEM((CHUNK + 64,), jnp.int32),
            pltpu.VMEM((CHUNK + 64,), jnp.int32),
        ],
    )(sims_flat)
    return pl.kernel(
        _sc_merge_body,
        out_type=jax.ShapeDtypeStruct((K, D), jnp.float32),
        mesh=mesh,
        compiler_params=pltpu.CompilerParams(needs_layout_passes=False),
        scratch_types=[
            pltpu.VMEM((NCAND,), jnp.int32),
            pltpu.VMEM((NCAND,), jnp.int32),
            pltpu.VMEM((4096,), jnp.int32),
            pltpu.VMEM((256,), jnp.int32),
            pltpu.VMEM((NCAND + 64,), jnp.int32),
            pltpu.VMEM((NCAND + 64,), jnp.int32),
            pltpu.VMEM((K,), jnp.int32),
            pltpu.VMEM((K,), jnp.int32),
            pltpu.VMEM((K,), jnp.int32),
            pltpu.VMEM((K, D), jnp.float32),
            pltpu.SemaphoreType.DMA,
        ],
    )(ckeys, cgidx, memory_tensor)


def _topk_body(sims_ref, idx_ref):
    sims = sims_ref[...]
    r = jax.lax.broadcasted_iota(jnp.int32, (ROWS, 128), 0)
    c = jax.lax.broadcasted_iota(jnp.int32, (ROWS, 128), 1)
    flat = r * 128 + c

    def body(k, s):
        m = jnp.max(s)
        i = jnp.min(jnp.where(s == m, flat, jnp.int32(2**30)))
        idx_ref[k] = i
        return jnp.where(flat == i, NEG, s)

    jax.lax.fori_loop(0, K, body, sims)


def _topk_idx(sims2d):
    return pl.pallas_call(
        _topk_body,
        out_specs=pl.BlockSpec(memory_space=pltpu.SMEM),
        out_shape=jax.ShapeDtypeStruct((K,), jnp.int32),
    )(sims2d)


def _gather_body(idx_ref, mem_ref, out_ref):
    out_ref[...] = mem_ref[...]


def _gather_rows(idx, memory_tensor):
    grid_spec = pltpu.PrefetchScalarGridSpec(
        num_scalar_prefetch=1,
        grid=(K,),
        in_specs=[pl.BlockSpec((1, 1, D), lambda i, idx_ref: (idx_ref[i], 0, 0))],
        out_specs=pl.BlockSpec((1, 1, D), lambda i, idx_ref: (i, 0, 0)),
    )
    out = pl.pallas_call(
        _gather_body,
        grid_spec=grid_spec,
        out_shape=jax.ShapeDtypeStruct((K, 1, D), jnp.float32),
    )(idx, memory_tensor.reshape(M, 1, D))
    return out.reshape(K, D)


def _layer_norm_in(x, w, b):
    m = jnp.mean(x, axis=-1, keepdims=True)
    d = x - m
    v = jnp.mean(d * d, axis=-1, keepdims=True)
    return d * (1.0 / jnp.sqrt(v + 1e-5)) * w + b


def _tfm_body(x0_ref, wi_ref, bi_ref, wo_ref, bo_ref, w1_ref, b1_ref,
              w2_ref, b2_ref, l1w_ref, l1b_ref, l2w_ref, l2b_ref, out_ref):
    x = x0_ref[...]
    inv_sqrt = 1.0 / (DH ** 0.5)
    for l in range(L):
        qkv = jnp.dot(x, wi_ref[l], preferred_element_type=jnp.float32) + bi_ref[l]
        heads = []
        for h in range(H):
            q = qkv[:, h * DH:(h + 1) * DH]
            k = qkv[:, D + h * DH:D + (h + 1) * DH]
            v = qkv[:, 2 * D + h * DH:2 * D + (h + 1) * DH]
            scores = jax.lax.dot_general(
                q, k, (((1,), (1,)), ((), ())),
                preferred_element_type=jnp.float32) * inv_sqrt
            mx = jnp.max(scores, axis=-1, keepdims=True)
            e = jnp.exp(scores - mx)
            attn = e / jnp.sum(e, axis=-1, keepdims=True)
            heads.append(jnp.dot(attn, v, preferred_element_type=jnp.float32))
        a = jnp.concatenate(heads, axis=1)
        a = jnp.dot(a, wo_ref[l], preferred_element_type=jnp.float32) + bo_ref[l]
        x = _layer_norm_in(x + a, l1w_ref[l], l1b_ref[l])
        ff = jnp.maximum(jnp.dot(x, w1_ref[l], preferred_element_type=jnp.float32) + b1_ref[l], 0.0)
        ff = jnp.dot(ff, w2_ref[l], preferred_element_type=jnp.float32) + b2_ref[l]
        x = _layer_norm_in(x + ff, l2w_ref[l], l2b_ref[l])
    out_ref[...] = x


def _transformer(x0, wi_t, bi, wo_t, bo, w1_t, b1, w2_t, b2, l1w, l1b, l2w, l2b):
    return pl.pallas_call(
        _tfm_body,
        out_shape=jax.ShapeDtypeStruct((K, D), jnp.float32),
    )(x0, wi_t, bi, wo_t, bo, w1_t, b1, w2_t, b2, l1w, l1b, l2w, l2b)


def kernel(current_context, memory_tensor, W_in, b_in, W_out, b_out,
           W1, b1, W2, b2, ln1_w, ln1_b, ln2_w, ln2_b, top_k):
    ctx_col = current_context.reshape(D, 1)
    sims = _compute_sims(memory_tensor, ctx_col)
    x0 = _sc_topk_gather(sims.reshape(M_PAD), memory_tensor)

    wi_t = jnp.transpose(W_in, (0, 2, 1))
    wo_t = jnp.transpose(W_out, (0, 2, 1))
    w1_t = jnp.transpose(W1, (0, 2, 1))
    w2_t = jnp.transpose(W2, (0, 2, 1))
    bi = b_in.reshape(L, 1, 3 * D)
    bo = b_out.reshape(L, 1, D)
    b1r = b1.reshape(L, 1, FF)
    b2r = b2.reshape(L, 1, D)
    l1w = ln1_w.reshape(L, 1, D)
    l1b = ln1_b.reshape(L, 1, D)
    l2w = ln2_w.reshape(L, 1, D)
    l2b = ln2_b.reshape(L, 1, D)
    return _transformer(x0, wi_t, bi, wo_t, bo, w1_t, b1r, w2_t, b2r,
                        l1w, l1b, l2w, l2b)


# speedup vs baseline: 1.1767x; 1.0594x over previous
import functools

import jax
import jax.numpy as jnp
from jax import lax
from jax.experimental import pallas as pl
from jax.experimental.pallas import tpu as pltpu
from jax.experimental.pallas import tpu_sc as plsc

D = 128
M = 100000
K = 64
L = 4
H = 8
DH = 16
FF = 512

MV_GRID = 4
MV_BLOCK = 25088
M_PAD = MV_GRID * MV_BLOCK
ROWS = M_PAD // 128

NEG = float("-inf")

NW = 32
CHUNK = M_PAD // NW
CVREGS = CHUNK // 16
NCAND = NW * K
MVREGS = NCAND // 16


def _sims_body(mem_ref, ctx_ref, out_ref):
    i = pl.program_id(0)
    s = jnp.dot(mem_ref[...], ctx_ref[...], preferred_element_type=jnp.float32)
    row = jax.lax.broadcasted_iota(jnp.int32, (MV_BLOCK, 1), 0) + i * MV_BLOCK
    out_ref[...] = jnp.where(row < M, s, NEG)


def _compute_sims(memory_tensor, ctx_col):
    return pl.pallas_call(
        _sims_body,
        grid=(MV_GRID,),
        in_specs=[
            pl.BlockSpec((MV_BLOCK, D), lambda i: (i, 0)),
            pl.BlockSpec((D, 1), lambda i: (0, 0)),
        ],
        out_specs=pl.BlockSpec((MV_BLOCK, 1), lambda i: (i, 0)),
        out_shape=jax.ShapeDtypeStruct((M_PAD, 1), jnp.float32),
    )(memory_tensor, ctx_col)


_SIGN_INT = -(2 ** 31)


def _f32_key(v):
    u = plsc.bitcast(v, jnp.int32)
    return jnp.where(u < 0, jnp.int32(-1) - (u + jnp.int32(_SIGN_INT)), u)


def _select_threshold(keys_ref, gidx_ref, nvregs, hist_v, tot_v, ck_v, cg_v, k):
    sign = jnp.int32(_SIGN_INT)
    lane = lax.iota(jnp.int32, 16)
    ones = jnp.ones((16,), jnp.int32)
    zeros = jnp.zeros((16,), jnp.int32)

    def zh(j, _):
        for u in range(8):
            hist_v[pl.ds(j * 128 + u * 16, 16)] = zeros
        return 0

    lax.fori_loop(0, 32, zh, 0)

    def hpass(j, _):
        for u in range(4):
            kv = keys_ref[pl.ds(j * 64 + u * 16, 16)]
            b = lax.shift_right_logical(kv + sign, jnp.int32(24))
            plsc.addupdate_scatter(hist_v, [lane * 256 + b], ones)
        return 0

    lax.fori_loop(0, nvregs // 4, hpass, 0)

    for g in range(16):
        acc = zeros
        for l in range(16):
            acc = acc + hist_v[pl.ds(l * 256 + g * 16, 16)]
        tot_v[pl.ds(g * 16, 16)] = acc

    run = jnp.int32(0)
    b_star = jnp.int32(0)
    found = jnp.int32(0)
    for g in range(15, -1, -1):
        tv = tot_v[pl.ds(g * 16, 16)]
        rv = lax.rev(tv, (0,))
        cs = plsc.cumsum(rv)
        m = (run + cs) >= k
        mi = jnp.where(m, 1, 0).astype(jnp.int32)
        any_m = jnp.sum(mi)
        mf = m & (plsc.cumsum(mi) == 1)
        i_star = jnp.sum(jnp.where(mf, lane, 0))
        b_g = g * 16 + 15 - i_star
        hit = (found == 0) & (any_m > 0)
        b_star = jnp.where(hit, b_g, b_star)
        found = jnp.where(hit, 1, found)
        run = run + jnp.sum(tv)

    ts8 = b_star * jnp.int32(1 << 24) + sign

    def cpass(j, off):
        for u in range(4):
            kv = keys_ref[pl.ds(j * 64 + u * 16, 16)]
            gv = gidx_ref[pl.ds(j * 64 + u * 16, 16)]
            m = kv >= ts8
            mi = jnp.where(m, 1, 0).astype(jnp.int32)
            pos = off + plsc.cumsum(mi) - mi
            plsc.store_scatter(ck_v, [pos], kv, mask=m)
            plsc.store_scatter(cg_v, [pos], gv, mask=m)
            off = off + jnp.sum(mi)
        return off

    nc = lax.fori_loop(0, nvregs // 4, cpass, jnp.int32(0))

    pad = jnp.broadcast_to(sign, (16,))
    for u in range(4):
        ck_v[pl.ds(nc + u * 16, 16)] = pad
    nv4 = lax.shift_right_logical(nc + 63, jnp.int32(6))

    def one_bit(_, carry):
        p, bit = carry
        t = p + bit
        ts = t + sign

        def cnt(j, acc):
            for u in range(4):
                kv = ck_v[pl.ds(j * 64 + u * 16, 16)]
                acc = acc + jnp.where(kv >= ts, 1, 0).astype(jnp.int32)
            return acc

        c = jnp.sum(lax.fori_loop(0, nv4, cnt, zeros))
        p = jnp.where(c >= k, t, p)
        return p, lax.shift_right_logical(bit, jnp.int32(1))

    p0 = b_star * jnp.int32(1 << 24)
    p, _ = lax.fori_loop(0, 24, one_bit, (p0, jnp.int32(1 << 23)))
    return p + sign, nc


def _sc_local_body(sims_hbm, ckeys_hbm, cgidx_hbm,
                   chunk_v, keys_v, gidx_v, hist_v, tot_v,
                   ck_v, cg_v, okey_v, ogid_v):
    wid = lax.axis_index("s") * 2 + lax.axis_index("c")
    base = wid * CHUNK
    pltpu.sync_copy(sims_hbm.at[pl.ds(base, CHUNK)], chunk_v)
    lane = lax.iota(jnp.int32, 16)

    def xform(j, _):
        for u in range(4):
            o = j * 64 + u * 16
            keys_v[pl.ds(o, 16)] = _f32_key(chunk_v[pl.ds(o, 16)])
            gidx_v[pl.ds(o, 16)] = base + o + lane
        return 0

    lax.fori_loop(0, CVREGS // 4, xform, 0)
    t_loc, nc = _select_threshold(keys_v, gidx_v, CVREGS, hist_v, tot_v,
                                  ck_v, cg_v, K)
    nv4 = lax.shift_right_logical(nc + 63, jnp.int32(6))

    def extract(j, off):
        for u in range(4):
            o = j * 64 + u * 16
            kv = ck_v[pl.ds(o, 16)]
            gv = cg_v[pl.ds(o, 16)]
            m = kv >= t_loc
            mi = jnp.where(m, 1, 0).astype(jnp.int32)
            pos = off + plsc.cumsum(mi) - mi
            plsc.store_scatter(okey_v, [pos], kv, mask=m)
            plsc.store_scatter(ogid_v, [pos], gv, mask=m)
            off = off + jnp.sum(mi)
        return off

    lax.fori_loop(0, nv4, extract, jnp.int32(0))
    pltpu.sync_copy(okey_v.at[pl.ds(0, K)], ckeys_hbm.at[pl.ds(wid * K, K)])
    pltpu.sync_copy(ogid_v.at[pl.ds(0, K)], cgidx_hbm.at[pl.ds(wid * K, K)])


def _sc_merge_body(ckeys_hbm, cgidx_hbm, mem_hbm, out_hbm,
                   k2_v, g2_v, hist_v, tot_v, ck_v, cg_v,
                   skey_v, sgid_v, sidx_v, rows_v, sem):
    wid = lax.axis_index("s") * 2 + lax.axis_index("c")

    @pl.when(wid == 0)
    def _():
        pltpu.sync_copy(ckeys_hbm, k2_v)
        pltpu.sync_copy(cgidx_hbm, g2_v)
        t_g, nc = _select_threshold(k2_v, g2_v, MVREGS, hist_v, tot_v,
                                    ck_v, cg_v, K)
        nv4 = lax.shift_right_logical(nc + 63, jnp.int32(6))

        def cnt_gt(j, acc):
            for u in range(4):
                kv = ck_v[pl.ds(j * 64 + u * 16, 16)]
                acc = acc + jnp.where(kv > t_g, 1, 0).astype(jnp.int32)
            return acc

        n_gt = jnp.sum(lax.fori_loop(0, nv4, cnt_gt, jnp.zeros((16,), jnp.int32)))
        r = K - n_gt

        def extract(j, carry):
            off, eq_seen = carry
            for u in range(4):
                o = j * 64 + u * 16
                kv = ck_v[pl.ds(o, 16)]
                gv = cg_v[pl.ds(o, 16)]
                m_gt = kv > t_g
                m_eq = kv == t_g
                ei = jnp.where(m_eq, 1, 0).astype(jnp.int32)
                eq_rank = eq_seen + plsc.cumsum(ei) - ei
                m = m_gt | (m_eq & (eq_rank < r))
                mi = jnp.where(m, 1, 0).astype(jnp.int32)
                pos = off + plsc.cumsum(mi) - mi
                plsc.store_scatter(skey_v, [pos], kv, mask=m)
                plsc.store_scatter(sgid_v, [pos], gv, mask=m)
                off = off + jnp.sum(mi)
                eq_seen = eq_seen + jnp.sum(ei)
            return off, eq_seen

        lax.fori_loop(0, nv4, extract, (jnp.int32(0), jnp.int32(0)))

        kvs = [skey_v[pl.ds(v * 16, 16)] for v in range(4)]
        gvs = [sgid_v[pl.ds(v * 16, 16)] for v in range(4)]

        def rank_step(j, ranks):
            jv = jnp.broadcast_to(j, (16,))
            kj = plsc.load_gather(skey_v, [jv])
            gj = plsc.load_gather(sgid_v, [jv])
            out = []
            for v in range(4):
                pre = (kj > kvs[v]) | ((kj == kvs[v]) & (gj < gvs[v]))
                out.append(ranks[v] + jnp.where(pre, 1, 0).astype(jnp.int32))
            return tuple(out)

        ranks = lax.fori_loop(0, K, rank_step,
                              tuple(jnp.zeros((16,), jnp.int32) for _ in range(4)))
        for v in range(4):
            plsc.store_scatter(sidx_v, [ranks[v]], gvs[v])

        pltpu.async_copy(mem_hbm.at[sidx_v], rows_v, sem).wait()
        pltpu.sync_copy(rows_v, out_hbm)


def _sc_topk_gather(sims_flat, memory_tensor):
    mesh = plsc.VectorSubcoreMesh(core_axis_name="c", subcore_axis_name="s")
    ckeys, cgidx = pl.kernel(
        _sc_local_body,
        out_type=(jax.ShapeDtypeStruct((NCAND,), jnp.int32),
                  jax.ShapeDtypeStruct((NCAND,), jnp.int32)),
        mesh=mesh,
        compiler_params=pltpu.CompilerParams(needs_layout_passes=False),
        scratch_types=[
            pltpu.VMEM((CHUNK,), jnp.float32),
            pltpu.VMEM((CHUNK,), jnp.int32),
            pltpu.VMEM((CHUNK,), jnp.int32),
            pltpu.VMEM((4096,), jnp.int32),
            pltpu.VMEM((256,), jnp.int32),
            pltpu.VMEM((CHUNK + 64,), jnp.int32),
            pltpu.VMEM((CHUNK + 64,), jnp.int32),
            pltpu.VMEM((CHUNK + 64,), jnp.int32),
            pltpu.VMEM((CHUNK + 64,), jnp.int32),
        ],
    )(sims_flat)
    return pl.kernel(
        _sc_merge_body,
        out_type=jax.ShapeDtypeStruct((K, D), jnp.float32),
        mesh=mesh,
        compiler_params=pltpu.CompilerParams(needs_layout_passes=False),
        scratch_types=[
            pltpu.VMEM((NCAND,), jnp.int32),
            pltpu.VMEM((NCAND,), jnp.int32),
            pltpu.VMEM((4096,), jnp.int32),
            pltpu.VMEM((256,), jnp.int32),
            pltpu.VMEM((NCAND + 64,), jnp.int32),
            pltpu.VMEM((NCAND + 64,), jnp.int32),
            pltpu.VMEM((K,), jnp.int32),
            pltpu.VMEM((K,), jnp.int32),
            pltpu.VMEM((K,), jnp.int32),
            pltpu.VMEM((K, D), jnp.float32),
            pltpu.SemaphoreType.DMA,
        ],
    )(ckeys, cgidx, memory_tensor)


def _topk_body(sims_ref, idx_ref):
    sims = sims_ref[...]
    r = jax.lax.broadcasted_iota(jnp.int32, (ROWS, 128), 0)
    c = jax.lax.broadcasted_iota(jnp.int32, (ROWS, 128), 1)
    flat = r * 128 + c

    def body(k, s):
        m = jnp.max(s)
        i = jnp.min(jnp.where(s == m, flat, jnp.int32(2**30)))
        idx_ref[k] = i
        return jnp.where(flat == i, NEG, s)

    jax.lax.fori_loop(0, K, body, sims)


def _topk_idx(sims2d):
    return pl.pallas_call(
        _topk_body,
        out_specs=pl.BlockSpec(memory_space=pltpu.SMEM),
        out_shape=jax.ShapeDtypeStruct((K,), jnp.int32),
    )(sims2d)


def _gather_body(idx_ref, mem_ref, out_ref):
    out_ref[...] = mem_ref[...]


def _gather_rows(idx, memory_tensor):
    grid_spec = pltpu.PrefetchScalarGridSpec(
        num_scalar_prefetch=1,
        grid=(K,),
        in_specs=[pl.BlockSpec((1, 1, D), lambda i, idx_ref: (idx_ref[i], 0, 0))],
        out_specs=pl.BlockSpec((1, 1, D), lambda i, idx_ref: (i, 0, 0)),
    )
    out = pl.pallas_call(
        _gather_body,
        grid_spec=grid_spec,
        out_shape=jax.ShapeDtypeStruct((K, 1, D), jnp.float32),
    )(idx, memory_tensor.reshape(M, 1, D))
    return out.reshape(K, D)


def _layer_norm_in(x, w, b):
    m = jnp.mean(x, axis=-1, keepdims=True)
    d = x - m
    v = jnp.mean(d * d, axis=-1, keepdims=True)
    return d * (1.0 / jnp.sqrt(v + 1e-5)) * w + b


def _tfm_body(x0_ref, wi_ref, bi_ref, wo_ref, bo_ref, w1_ref, b1_ref,
              w2_ref, b2_ref, l1w_ref, l1b_ref, l2w_ref, l2b_ref, out_ref):
    x = x0_ref[...]
    inv_sqrt = 1.0 / (DH ** 0.5)
    SH = K * H
    r6 = jax.lax.broadcasted_iota(jnp.int32, (SH, SH), 0) // K
    c6 = jax.lax.broadcasted_iota(jnp.int32, (SH, SH), 1) // K
    blk = r6 == c6
    for l in range(L):
        qkv = jnp.dot(x, wi_ref[l], preferred_element_type=jnp.float32) + bi_ref[l]
        qh = jnp.concatenate([qkv[:, h * DH:(h + 1) * DH] for h in range(H)], axis=0)
        kh = jnp.concatenate([qkv[:, D + h * DH:D + (h + 1) * DH] for h in range(H)], axis=0)
        vh = jnp.concatenate([qkv[:, 2 * D + h * DH:2 * D + (h + 1) * DH] for h in range(H)], axis=0)
        scores = jax.lax.dot_general(
            qh, kh, (((1,), (1,)), ((), ())),
            preferred_element_type=jnp.float32) * inv_sqrt
        scores = jnp.where(blk, scores, NEG)
        mx = jnp.max(scores, axis=-1, keepdims=True)
        e = jnp.exp(scores - mx)
        attn = e / jnp.sum(e, axis=-1, keepdims=True)
        oh = jnp.dot(attn, vh, preferred_element_type=jnp.float32)
        a = jnp.concatenate([oh[h * K:(h + 1) * K, :] for h in range(H)], axis=1)
        a = jnp.dot(a, wo_ref[l], preferred_element_type=jnp.float32) + bo_ref[l]
        x = _layer_norm_in(x + a, l1w_ref[l], l1b_ref[l])
        ff = jnp.maximum(jnp.dot(x, w1_ref[l], preferred_element_type=jnp.float32) + b1_ref[l], 0.0)
        ff = jnp.dot(ff, w2_ref[l], preferred_element_type=jnp.float32) + b2_ref[l]
        x = _layer_norm_in(x + ff, l2w_ref[l], l2b_ref[l])
    out_ref[...] = x


def _transformer(x0, wi_t, bi, wo_t, bo, w1_t, b1, w2_t, b2, l1w, l1b, l2w, l2b):
    return pl.pallas_call(
        _tfm_body,
        out_shape=jax.ShapeDtypeStruct((K, D), jnp.float32),
    )(x0, wi_t, bi, wo_t, bo, w1_t, b1, w2_t, b2, l1w, l1b, l2w, l2b)


def kernel(current_context, memory_tensor, W_in, b_in, W_out, b_out,
           W1, b1, W2, b2, ln1_w, ln1_b, ln2_w, ln2_b, top_k):
    ctx_col = current_context.reshape(D, 1)
    sims = _compute_sims(memory_tensor, ctx_col)
    x0 = _sc_topk_gather(sims.reshape(M_PAD), memory_tensor)

    wi_t = jnp.transpose(W_in, (0, 2, 1))
    wo_t = jnp.transpose(W_out, (0, 2, 1))
    w1_t = jnp.transpose(W1, (0, 2, 1))
    w2_t = jnp.transpose(W2, (0, 2, 1))
    bi = b_in.reshape(L, 1, 3 * D)
    bo = b_out.reshape(L, 1, D)
    b1r = b1.reshape(L, 1, FF)
    b2r = b2.reshape(L, 1, D)
    l1w = ln1_w.reshape(L, 1, D)
    l1b = ln1_b.reshape(L, 1, D)
    l2w = ln2_w.reshape(L, 1, D)
    l2b = ln2_b.reshape(L, 1, D)
    return _transformer(x0, wi_t, bi, wo_t, bo, w1_t, b1r, w2_t, b2r,
                        l1w, l1b, l2w, l2b)


# speedup vs baseline: 1.1930x; 1.0139x over previous
import functools

import jax
import jax.numpy as jnp
from jax import lax
from jax.experimental import pallas as pl
from jax.experimental.pallas import tpu as pltpu
from jax.experimental.pallas import tpu_sc as plsc

D = 128
M = 100000
K = 64
L = 4
H = 8
DH = 16
FF = 512

MV_GRID = 4
MV_BLOCK = 25088
M_PAD = MV_GRID * MV_BLOCK
ROWS = M_PAD // 128

NEG = float("-inf")

NW = 32
CHUNK = M_PAD // NW
CVREGS = CHUNK // 16
NCAND = NW * K
MVREGS = NCAND // 16


def _sims_body(mem_ref, ctx_ref, out_ref):
    i = pl.program_id(0)
    s = jnp.dot(mem_ref[...], ctx_ref[...], preferred_element_type=jnp.float32)
    row = jax.lax.broadcasted_iota(jnp.int32, (MV_BLOCK, 1), 0) + i * MV_BLOCK
    out_ref[...] = jnp.where(row < M, s, NEG)


def _compute_sims(memory_tensor, ctx_col):
    return pl.pallas_call(
        _sims_body,
        grid=(MV_GRID,),
        in_specs=[
            pl.BlockSpec((MV_BLOCK, D), lambda i: (i, 0)),
            pl.BlockSpec((D, 1), lambda i: (0, 0)),
        ],
        out_specs=pl.BlockSpec((MV_BLOCK, 1), lambda i: (i, 0)),
        out_shape=jax.ShapeDtypeStruct((M_PAD, 1), jnp.float32),
    )(memory_tensor, ctx_col)


_SIGN_INT = -(2 ** 31)


def _f32_key(v):
    u = plsc.bitcast(v, jnp.int32)
    return jnp.where(u < 0, jnp.int32(-1) - (u + jnp.int32(_SIGN_INT)), u)


def _select_threshold(keys_ref, gidx_ref, nvregs, hist_v, tot_v, ck_v, cg_v, k):
    sign = jnp.int32(_SIGN_INT)
    lane = lax.iota(jnp.int32, 16)
    ones = jnp.ones((16,), jnp.int32)
    zeros = jnp.zeros((16,), jnp.int32)

    def zh(j, _):
        for u in range(8):
            hist_v[pl.ds(j * 128 + u * 16, 16)] = zeros
        return 0

    lax.fori_loop(0, 32, zh, 0)

    def hpass(j, _):
        for u in range(4):
            kv = keys_ref[pl.ds(j * 64 + u * 16, 16)]
            b = lax.shift_right_logical(kv + sign, jnp.int32(24))
            plsc.addupdate_scatter(hist_v, [lane * 256 + b], ones)
        return 0

    lax.fori_loop(0, nvregs // 4, hpass, 0)

    for g in range(16):
        acc = zeros
        for l in range(16):
            acc = acc + hist_v[pl.ds(l * 256 + g * 16, 16)]
        tot_v[pl.ds(g * 16, 16)] = acc

    run = jnp.int32(0)
    b_star = jnp.int32(0)
    found = jnp.int32(0)
    for g in range(15, -1, -1):
        tv = tot_v[pl.ds(g * 16, 16)]
        rv = lax.rev(tv, (0,))
        cs = plsc.cumsum(rv)
        m = (run + cs) >= k
        mi = jnp.where(m, 1, 0).astype(jnp.int32)
        any_m = jnp.sum(mi)
        mf = m & (plsc.cumsum(mi) == 1)
        i_star = jnp.sum(jnp.where(mf, lane, 0))
        b_g = g * 16 + 15 - i_star
        hit = (found == 0) & (any_m > 0)
        b_star = jnp.where(hit, b_g, b_star)
        found = jnp.where(hit, 1, found)
        run = run + jnp.sum(tv)

    ts8 = b_star * jnp.int32(1 << 24) + sign

    def cpass(j, off):
        for u in range(4):
            kv = keys_ref[pl.ds(j * 64 + u * 16, 16)]
            gv = gidx_ref[pl.ds(j * 64 + u * 16, 16)]
            m = kv >= ts8
            mi = jnp.where(m, 1, 0).astype(jnp.int32)
            pos = off + plsc.cumsum(mi) - mi
            plsc.store_scatter(ck_v, [pos], kv, mask=m)
            plsc.store_scatter(cg_v, [pos], gv, mask=m)
            off = off + jnp.sum(mi)
        return off

    nc = lax.fori_loop(0, nvregs // 4, cpass, jnp.int32(0))

    pad = jnp.broadcast_to(sign, (16,))
    for u in range(4):
        ck_v[pl.ds(nc + u * 16, 16)] = pad
    nv4 = lax.shift_right_logical(nc + 63, jnp.int32(6))

    def one_bit(_, carry):
        p, bit = carry
        t = p + bit
        ts = t + sign

        def cnt(j, acc):
            for u in range(4):
                kv = ck_v[pl.ds(j * 64 + u * 16, 16)]
                acc = acc + jnp.where(kv >= ts, 1, 0).astype(jnp.int32)
            return acc

        c = jnp.sum(lax.fori_loop(0, nv4, cnt, zeros))
        p = jnp.where(c >= k, t, p)
        return p, lax.shift_right_logical(bit, jnp.int32(1))

    p0 = b_star * jnp.int32(1 << 24)
    p, _ = lax.fori_loop(0, 24, one_bit, (p0, jnp.int32(1 << 23)))
    return p + sign, nc


def _sc_local_body(sims_hbm, ckeys_hbm, cgidx_hbm,
                   chunk_v, keys_v, gidx_v, hist_v, tot_v,
                   ck_v, cg_v, okey_v, ogid_v):
    wid = lax.axis_index("s") * 2 + lax.axis_index("c")
    base = wid * CHUNK
    pltpu.sync_copy(sims_hbm.at[pl.ds(base, CHUNK)], chunk_v)
    lane = lax.iota(jnp.int32, 16)

    def xform(j, _):
        for u in range(4):
            o = j * 64 + u * 16
            keys_v[pl.ds(o, 16)] = _f32_key(chunk_v[pl.ds(o, 16)])
            gidx_v[pl.ds(o, 16)] = base + o + lane
        return 0

    lax.fori_loop(0, CVREGS // 4, xform, 0)
    t_loc, nc = _select_threshold(keys_v, gidx_v, CVREGS, hist_v, tot_v,
                                  ck_v, cg_v, K)
    nv4 = lax.shift_right_logical(nc + 63, jnp.int32(6))

    def extract(j, off):
        for u in range(4):
            o = j * 64 + u * 16
            kv = ck_v[pl.ds(o, 16)]
            gv = cg_v[pl.ds(o, 16)]
            m = kv >= t_loc
            mi = jnp.where(m, 1, 0).astype(jnp.int32)
            pos = off + plsc.cumsum(mi) - mi
            plsc.store_scatter(okey_v, [pos], kv, mask=m)
            plsc.store_scatter(ogid_v, [pos], gv, mask=m)
            off = off + jnp.sum(mi)
        return off

    lax.fori_loop(0, nv4, extract, jnp.int32(0))
    pltpu.sync_copy(okey_v.at[pl.ds(0, K)], ckeys_hbm.at[pl.ds(wid * K, K)])
    pltpu.sync_copy(ogid_v.at[pl.ds(0, K)], cgidx_hbm.at[pl.ds(wid * K, K)])


def _sc_merge_body(ckeys_hbm, cgidx_hbm, mem_hbm, out_hbm,
                   k2_v, g2_v, hist_v, tot_v, ck_v, cg_v,
                   skey_v, sgid_v, sidx_v, rows_v, sem):
    wid = lax.axis_index("s") * 2 + lax.axis_index("c")

    @pl.when(wid == 0)
    def _():
        pltpu.sync_copy(ckeys_hbm, k2_v)
        pltpu.sync_copy(cgidx_hbm, g2_v)
        t_g, nc = _select_threshold(k2_v, g2_v, MVREGS, hist_v, tot_v,
                                    ck_v, cg_v, K)
        nv4 = lax.shift_right_logical(nc + 63, jnp.int32(6))

        def cnt_gt(j, acc):
            for u in range(4):
                kv = ck_v[pl.ds(j * 64 + u * 16, 16)]
                acc = acc + jnp.where(kv > t_g, 1, 0).astype(jnp.int32)
            return acc

        n_gt = jnp.sum(lax.fori_loop(0, nv4, cnt_gt, jnp.zeros((16,), jnp.int32)))
        r = K - n_gt

        def extract(j, carry):
            off, eq_seen = carry
            for u in range(4):
                o = j * 64 + u * 16
                kv = ck_v[pl.ds(o, 16)]
                gv = cg_v[pl.ds(o, 16)]
                m_gt = kv > t_g
                m_eq = kv == t_g
                ei = jnp.where(m_eq, 1, 0).astype(jnp.int32)
                eq_rank = eq_seen + plsc.cumsum(ei) - ei
                m = m_gt | (m_eq & (eq_rank < r))
                mi = jnp.where(m, 1, 0).astype(jnp.int32)
                pos = off + plsc.cumsum(mi) - mi
                plsc.store_scatter(skey_v, [pos], kv, mask=m)
                plsc.store_scatter(sgid_v, [pos], gv, mask=m)
                off = off + jnp.sum(mi)
                eq_seen = eq_seen + jnp.sum(ei)
            return off, eq_seen

        lax.fori_loop(0, nv4, extract, (jnp.int32(0), jnp.int32(0)))

        kvs = [skey_v[pl.ds(v * 16, 16)] for v in range(4)]
        gvs = [sgid_v[pl.ds(v * 16, 16)] for v in range(4)]

        def rank_step(j, ranks):
            jv = jnp.broadcast_to(j, (16,))
            kj = plsc.load_gather(skey_v, [jv])
            gj = plsc.load_gather(sgid_v, [jv])
            out = []
            for v in range(4):
                pre = (kj > kvs[v]) | ((kj == kvs[v]) & (gj < gvs[v]))
                out.append(ranks[v] + jnp.where(pre, 1, 0).astype(jnp.int32))
            return tuple(out)

        ranks = lax.fori_loop(0, K, rank_step,
                              tuple(jnp.zeros((16,), jnp.int32) for _ in range(4)))
        for v in range(4):
            plsc.store_scatter(sidx_v, [ranks[v]], gvs[v])

        pltpu.async_copy(mem_hbm.at[sidx_v], rows_v, sem).wait()
        pltpu.sync_copy(rows_v, out_hbm)


NW1 = 16
CHUNK1 = M_PAD // NW1
CV1 = CHUNK1 // 16
NCAND1 = NW1 * K


def _sc_fused_body(sims_hbm, mem_hbm, out_hbm,
                   chunk_v, keys_v, gidx_v, hist_v, tot_v,
                   ck_v, cg_v, okey_v, ogid_v,
                   k2_v, g2_v, skey_v, sgid_v, sidx_v, rows_v,
                   shk_sh, shg_sh, sem):
    wid = lax.axis_index("s")
    base = wid * CHUNK1
    pltpu.sync_copy(sims_hbm.at[pl.ds(base, CHUNK1)], chunk_v)
    lane = lax.iota(jnp.int32, 16)

    def xform(j, _):
        for u in range(4):
            o = j * 64 + u * 16
            keys_v[pl.ds(o, 16)] = _f32_key(chunk_v[pl.ds(o, 16)])
            gidx_v[pl.ds(o, 16)] = base + o + lane
        return 0

    lax.fori_loop(0, CV1 // 4, xform, 0)
    t_loc, nc = _select_threshold(keys_v, gidx_v, CV1, hist_v, tot_v,
                                  ck_v, cg_v, K)
    nv4 = lax.shift_right_logical(nc + 63, jnp.int32(6))

    def extract(j, off):
        for u in range(4):
            o = j * 64 + u * 16
            kv = ck_v[pl.ds(o, 16)]
            gv = cg_v[pl.ds(o, 16)]
            m = kv >= t_loc
            mi = jnp.where(m, 1, 0).astype(jnp.int32)
            pos = off + plsc.cumsum(mi) - mi
            plsc.store_scatter(okey_v, [pos], kv, mask=m)
            plsc.store_scatter(ogid_v, [pos], gv, mask=m)
            off = off + jnp.sum(mi)
        return off

    lax.fori_loop(0, nv4, extract, jnp.int32(0))
    pltpu.sync_copy(okey_v.at[pl.ds(0, K)], shk_sh.at[pl.ds(wid * K, K)])
    pltpu.sync_copy(ogid_v.at[pl.ds(0, K)], shg_sh.at[pl.ds(wid * K, K)])
    plsc.subcore_barrier()

    @pl.when(wid == 0)
    def _():
        pltpu.sync_copy(shk_sh, k2_v)
        pltpu.sync_copy(shg_sh, g2_v)
        t_g, nc2 = _select_threshold(k2_v, g2_v, NCAND1 // 16, hist_v, tot_v,
                                     ck_v, cg_v, K)
        nv4b = lax.shift_right_logical(nc2 + 63, jnp.int32(6))

        def cnt_gt(j, acc):
            for u in range(4):
                kv = ck_v[pl.ds(j * 64 + u * 16, 16)]
                acc = acc + jnp.where(kv > t_g, 1, 0).astype(jnp.int32)
            return acc

        n_gt = jnp.sum(lax.fori_loop(0, nv4b, cnt_gt, jnp.zeros((16,), jnp.int32)))
        r = K - n_gt

        def extract2(j, carry):
            off, eq_seen = carry
            for u in range(4):
                o = j * 64 + u * 16
                kv = ck_v[pl.ds(o, 16)]
                gv = cg_v[pl.ds(o, 16)]
                m_gt = kv > t_g
                m_eq = kv == t_g
                ei = jnp.where(m_eq, 1, 0).astype(jnp.int32)
                eq_rank = eq_seen + plsc.cumsum(ei) - ei
                m = m_gt | (m_eq & (eq_rank < r))
                mi = jnp.where(m, 1, 0).astype(jnp.int32)
                pos = off + plsc.cumsum(mi) - mi
                plsc.store_scatter(skey_v, [pos], kv, mask=m)
                plsc.store_scatter(sgid_v, [pos], gv, mask=m)
                off = off + jnp.sum(mi)
                eq_seen = eq_seen + jnp.sum(ei)
            return off, eq_seen

        lax.fori_loop(0, nv4b, extract2, (jnp.int32(0), jnp.int32(0)))

        kvs = [skey_v[pl.ds(v * 16, 16)] for v in range(4)]
        gvs = [sgid_v[pl.ds(v * 16, 16)] for v in range(4)]

        def rank_step(j, ranks):
            jv = jnp.broadcast_to(j, (16,))
            kj = plsc.load_gather(skey_v, [jv])
            gj = plsc.load_gather(sgid_v, [jv])
            out = []
            for v in range(4):
                pre = (kj > kvs[v]) | ((kj == kvs[v]) & (gj < gvs[v]))
                out.append(ranks[v] + jnp.where(pre, 1, 0).astype(jnp.int32))
            return tuple(out)

        ranks = lax.fori_loop(0, K, rank_step,
                              tuple(jnp.zeros((16,), jnp.int32) for _ in range(4)))
        for v in range(4):
            plsc.store_scatter(sidx_v, [ranks[v]], gvs[v])

        pltpu.async_copy(mem_hbm.at[sidx_v], rows_v, sem).wait()
        pltpu.sync_copy(rows_v, out_hbm)


def _sc_topk_gather_fused(sims_flat, memory_tensor):
    mesh = plsc.VectorSubcoreMesh(core_axis_name="c", subcore_axis_name="s",
                                  num_cores=1)
    return pl.kernel(
        _sc_fused_body,
        out_type=jax.ShapeDtypeStruct((K, D), jnp.float32),
        mesh=mesh,
        compiler_params=pltpu.CompilerParams(needs_layout_passes=False),
        scratch_types=[
            pltpu.VMEM((CHUNK1,), jnp.float32),
            pltpu.VMEM((CHUNK1,), jnp.int32),
            pltpu.VMEM((CHUNK1,), jnp.int32),
            pltpu.VMEM((4096,), jnp.int32),
            pltpu.VMEM((256,), jnp.int32),
            pltpu.VMEM((CHUNK1 + 64,), jnp.int32),
            pltpu.VMEM((CHUNK1 + 64,), jnp.int32),
            pltpu.VMEM((CHUNK1 + 64,), jnp.int32),
            pltpu.VMEM((CHUNK1 + 64,), jnp.int32),
            pltpu.VMEM((NCAND1,), jnp.int32),
            pltpu.VMEM((NCAND1,), jnp.int32),
            pltpu.VMEM((K,), jnp.int32),
            pltpu.VMEM((K,), jnp.int32),
            pltpu.VMEM((K,), jnp.int32),
            pltpu.VMEM((K, D), jnp.float32),
            pltpu.VMEM_SHARED((NCAND1,), jnp.int32),
            pltpu.VMEM_SHARED((NCAND1,), jnp.int32),
            pltpu.SemaphoreType.DMA,
        ],
    )(sims_flat, memory_tensor)


def _sc_topk_gather(sims_flat, memory_tensor):
    mesh = plsc.VectorSubcoreMesh(core_axis_name="c", subcore_axis_name="s")
    ckeys, cgidx = pl.kernel(
        _sc_local_body,
        out_type=(jax.ShapeDtypeStruct((NCAND,), jnp.int32),
                  jax.ShapeDtypeStruct((NCAND,), jnp.int32)),
        mesh=mesh,
        compiler_params=pltpu.CompilerParams(needs_layout_passes=False),
        scratch_types=[
            pltpu.VMEM((CHUNK,), jnp.float32),
            pltpu.VMEM((CHUNK,), jnp.int32),
            pltpu.VMEM((CHUNK,), jnp.int32),
            pltpu.VMEM((4096,), jnp.int32),
            pltpu.VMEM((256,), jnp.int32),
            pltpu.VMEM((CHUNK + 64,), jnp.int32),
            pltpu.VMEM((CHUNK + 64,), jnp.int32),
            pltpu.VMEM((CHUNK + 64,), jnp.int32),
            pltpu.VMEM((CHUNK + 64,), jnp.int32),
        ],
    )(sims_flat)
    return pl.kernel(
        _sc_merge_body,
        out_type=jax.ShapeDtypeStruct((K, D), jnp.float32),
        mesh=mesh,
        compiler_params=pltpu.CompilerParams(needs_layout_passes=False),
        scratch_types=[
            pltpu.VMEM((NCAND,), jnp.int32),
            pltpu.VMEM((NCAND,), jnp.int32),
            pltpu.VMEM((4096,), jnp.int32),
            pltpu.VMEM((256,), jnp.int32),
            pltpu.VMEM((NCAND + 64,), jnp.int32),
            pltpu.VMEM((NCAND + 64,), jnp.int32),
            pltpu.VMEM((K,), jnp.int32),
            pltpu.VMEM((K,), jnp.int32),
            pltpu.VMEM((K,), jnp.int32),
            pltpu.VMEM((K, D), jnp.float32),
            pltpu.SemaphoreType.DMA,
        ],
    )(ckeys, cgidx, memory_tensor)


def _topk_body(sims_ref, idx_ref):
    sims = sims_ref[...]
    r = jax.lax.broadcasted_iota(jnp.int32, (ROWS, 128), 0)
    c = jax.lax.broadcasted_iota(jnp.int32, (ROWS, 128), 1)
    flat = r * 128 + c

    def body(k, s):
        m = jnp.max(s)
        i = jnp.min(jnp.where(s == m, flat, jnp.int32(2**30)))
        idx_ref[k] = i
        return jnp.where(flat == i, NEG, s)

    jax.lax.fori_loop(0, K, body, sims)


def _topk_idx(sims2d):
    return pl.pallas_call(
        _topk_body,
        out_specs=pl.BlockSpec(memory_space=pltpu.SMEM),
        out_shape=jax.ShapeDtypeStruct((K,), jnp.int32),
    )(sims2d)


def _gather_body(idx_ref, mem_ref, out_ref):
    out_ref[...] = mem_ref[...]


def _gather_rows(idx, memory_tensor):
    grid_spec = pltpu.PrefetchScalarGridSpec(
        num_scalar_prefetch=1,
        grid=(K,),
        in_specs=[pl.BlockSpec((1, 1, D), lambda i, idx_ref: (idx_ref[i], 0, 0))],
        out_specs=pl.BlockSpec((1, 1, D), lambda i, idx_ref: (i, 0, 0)),
    )
    out = pl.pallas_call(
        _gather_body,
        grid_spec=grid_spec,
        out_shape=jax.ShapeDtypeStruct((K, 1, D), jnp.float32),
    )(idx, memory_tensor.reshape(M, 1, D))
    return out.reshape(K, D)


def _layer_norm_in(x, w, b):
    m = jnp.mean(x, axis=-1, keepdims=True)
    d = x - m
    v = jnp.mean(d * d, axis=-1, keepdims=True)
    return d * (1.0 / jnp.sqrt(v + 1e-5)) * w + b


def _tfm_body(x0_ref, wi_ref, bi_ref, wo_ref, bo_ref, w1_ref, b1_ref,
              w2_ref, b2_ref, l1w_ref, l1b_ref, l2w_ref, l2b_ref, out_ref):
    x = x0_ref[...]
    inv_sqrt = 1.0 / (DH ** 0.5)
    SH = K * H
    r6 = jax.lax.broadcasted_iota(jnp.int32, (SH, SH), 0) // K
    c6 = jax.lax.broadcasted_iota(jnp.int32, (SH, SH), 1) // K
    blk = r6 == c6
    for l in range(L):
        qkv = jnp.dot(x, wi_ref[l], preferred_element_type=jnp.float32) + bi_ref[l]
        qh = jnp.concatenate([qkv[:, h * DH:(h + 1) * DH] for h in range(H)], axis=0)
        kh = jnp.concatenate([qkv[:, D + h * DH:D + (h + 1) * DH] for h in range(H)], axis=0)
        vh = jnp.concatenate([qkv[:, 2 * D + h * DH:2 * D + (h + 1) * DH] for h in range(H)], axis=0)
        scores = jax.lax.dot_general(
            qh, kh, (((1,), (1,)), ((), ())),
            preferred_element_type=jnp.float32) * inv_sqrt
        scores = jnp.where(blk, scores, NEG)
        mx = jnp.max(scores, axis=-1, keepdims=True)
        e = jnp.exp(scores - mx)
        attn = e / jnp.sum(e, axis=-1, keepdims=True)
        oh = jnp.dot(attn, vh, preferred_element_type=jnp.float32)
        a = jnp.concatenate([oh[h * K:(h + 1) * K, :] for h in range(H)], axis=1)
        a = jnp.dot(a, wo_ref[l], preferred_element_type=jnp.float32) + bo_ref[l]
        x = _layer_norm_in(x + a, l1w_ref[l], l1b_ref[l])
        ff = jnp.maximum(jnp.dot(x, w1_ref[l], preferred_element_type=jnp.float32) + b1_ref[l], 0.0)
        ff = jnp.dot(ff, w2_ref[l], preferred_element_type=jnp.float32) + b2_ref[l]
        x = _layer_norm_in(x + ff, l2w_ref[l], l2b_ref[l])
    out_ref[...] = x


def _transformer(x0, wi_t, bi, wo_t, bo, w1_t, b1, w2_t, b2, l1w, l1b, l2w, l2b):
    return pl.pallas_call(
        _tfm_body,
        out_shape=jax.ShapeDtypeStruct((K, D), jnp.float32),
    )(x0, wi_t, bi, wo_t, bo, w1_t, b1, w2_t, b2, l1w, l1b, l2w, l2b)


def kernel(current_context, memory_tensor, W_in, b_in, W_out, b_out,
           W1, b1, W2, b2, ln1_w, ln1_b, ln2_w, ln2_b, top_k):
    ctx_col = current_context.reshape(D, 1)
    sims = _compute_sims(memory_tensor, ctx_col)
    x0 = _sc_topk_gather_fused(sims.reshape(M_PAD), memory_tensor)

    wi_t = jnp.transpose(W_in, (0, 2, 1))
    wo_t = jnp.transpose(W_out, (0, 2, 1))
    w1_t = jnp.transpose(W1, (0, 2, 1))
    w2_t = jnp.transpose(W2, (0, 2, 1))
    bi = b_in.reshape(L, 1, 3 * D)
    bo = b_out.reshape(L, 1, D)
    b1r = b1.reshape(L, 1, FF)
    b2r = b2.reshape(L, 1, D)
    l1w = ln1_w.reshape(L, 1, D)
    l1b = ln1_b.reshape(L, 1, D)
    l2w = ln2_w.reshape(L, 1, D)
    l2b = ln2_b.reshape(L, 1, D)
    return _transformer(x0, wi_t, bi, wo_t, bo, w1_t, b1r, w2_t, b2r,
                        l1w, l1b, l2w, l2b)


# speedup vs baseline: 1.7605x; 1.4757x over previous
import functools

import jax
import jax.numpy as jnp
from jax import lax
from jax.experimental import pallas as pl
from jax.experimental.pallas import tpu as pltpu
from jax.experimental.pallas import tpu_sc as plsc

D = 128
M = 100000
K = 64
L = 4
H = 8
DH = 16
FF = 512

MV_GRID = 7
MV_BLOCK = 14336
M_PAD = MV_GRID * MV_BLOCK
ROWS = M_PAD // 128

NEG = float("-inf")

NW = 32
CHUNK = M_PAD // NW
CVREGS = CHUNK // 16
NCAND = NW * K
MVREGS = NCAND // 16


def _sims_body(mem_ref, ctx_ref, out_ref):
    i = pl.program_id(0)
    s = jax.lax.dot_general(ctx_ref[...], mem_ref[...], (((1,), (1,)), ((), ())),
                            preferred_element_type=jnp.float32)
    col = jax.lax.broadcasted_iota(jnp.int32, (1, MV_BLOCK), 1) + i * MV_BLOCK
    out_ref[...] = jnp.where(col < M, s, NEG).reshape(MV_BLOCK)


def _compute_sims(memory_tensor, ctx_row):
    return pl.pallas_call(
        _sims_body,
        grid=(MV_GRID,),
        in_specs=[
            pl.BlockSpec((MV_BLOCK, D), lambda i: (i, 0)),
            pl.BlockSpec((1, D), lambda i: (0, 0)),
        ],
        out_specs=pl.BlockSpec((MV_BLOCK,), lambda i: (i,)),
        out_shape=jax.ShapeDtypeStruct((M_PAD,), jnp.float32),
    )(memory_tensor, ctx_row)


_SIGN_INT = -(2 ** 31)


def _f32_key(v):
    u = plsc.bitcast(v, jnp.int32)
    return jnp.where(u < 0, jnp.int32(-1) - (u + jnp.int32(_SIGN_INT)), u)


def _select_threshold(keys_ref, gidx_ref, nvregs, hist_v, tot_v, ck_v, cg_v, k):
    sign = jnp.int32(_SIGN_INT)
    lane = lax.iota(jnp.int32, 16)
    ones = jnp.ones((16,), jnp.int32)
    zeros = jnp.zeros((16,), jnp.int32)

    def zh(j, _):
        for u in range(8):
            hist_v[pl.ds(j * 128 + u * 16, 16)] = zeros
        return 0

    lax.fori_loop(0, 32, zh, 0)

    def hpass(j, _):
        for u in range(4):
            kv = keys_ref[pl.ds(j * 64 + u * 16, 16)]
            b = lax.shift_right_logical(kv + sign, jnp.int32(24))
            plsc.addupdate_scatter(hist_v, [lane * 256 + b], ones)
        return 0

    lax.fori_loop(0, nvregs // 4, hpass, 0)

    for g in range(16):
        acc = zeros
        for l in range(16):
            acc = acc + hist_v[pl.ds(l * 256 + g * 16, 16)]
        tot_v[pl.ds(g * 16, 16)] = acc

    run = jnp.int32(0)
    b_star = jnp.int32(0)
    found = jnp.int32(0)
    for g in range(15, -1, -1):
        tv = tot_v[pl.ds(g * 16, 16)]
        rv = lax.rev(tv, (0,))
        cs = plsc.cumsum(rv)
        m = (run + cs) >= k
        mi = jnp.where(m, 1, 0).astype(jnp.int32)
        any_m = jnp.sum(mi)
        mf = m & (plsc.cumsum(mi) == 1)
        i_star = jnp.sum(jnp.where(mf, lane, 0))
        b_g = g * 16 + 15 - i_star
        hit = (found == 0) & (any_m > 0)
        b_star = jnp.where(hit, b_g, b_star)
        found = jnp.where(hit, 1, found)
        run = run + jnp.sum(tv)

    ts8 = b_star * jnp.int32(1 << 24) + sign

    def cpass(j, off):
        for u in range(4):
            kv = keys_ref[pl.ds(j * 64 + u * 16, 16)]
            gv = gidx_ref[pl.ds(j * 64 + u * 16, 16)]
            m = kv >= ts8
            mi = jnp.where(m, 1, 0).astype(jnp.int32)
            pos = off + plsc.cumsum(mi) - mi
            plsc.store_scatter(ck_v, [pos], kv, mask=m)
            plsc.store_scatter(cg_v, [pos], gv, mask=m)
            off = off + jnp.sum(mi)
        return off

    nc = lax.fori_loop(0, nvregs // 4, cpass, jnp.int32(0))

    pad = jnp.broadcast_to(sign, (16,))
    for u in range(4):
        ck_v[pl.ds(nc + u * 16, 16)] = pad
    nv4 = lax.shift_right_logical(nc + 63, jnp.int32(6))

    def one_bit(_, carry):
        p, bit = carry
        t = p + bit
        ts = t + sign

        def cnt(j, acc):
            for u in range(4):
                kv = ck_v[pl.ds(j * 64 + u * 16, 16)]
                acc = acc + jnp.where(kv >= ts, 1, 0).astype(jnp.int32)
            return acc

        c = jnp.sum(lax.fori_loop(0, nv4, cnt, zeros))
        p = jnp.where(c >= k, t, p)
        return p, lax.shift_right_logical(bit, jnp.int32(1))

    p0 = b_star * jnp.int32(1 << 24)
    p, _ = lax.fori_loop(0, 24, one_bit, (p0, jnp.int32(1 << 23)))
    return p + sign, nc


def _sc_local_body(sims_hbm, ckeys_hbm, cgidx_hbm,
                   chunk_v, keys_v, gidx_v, hist_v, tot_v,
                   ck_v, cg_v, okey_v, ogid_v):
    wid = lax.axis_index("s") * 2 + lax.axis_index("c")
    base = wid * CHUNK
    pltpu.sync_copy(sims_hbm.at[pl.ds(base, CHUNK)], chunk_v)
    lane = lax.iota(jnp.int32, 16)

    def xform(j, _):
        for u in range(4):
            o = j * 64 + u * 16
            keys_v[pl.ds(o, 16)] = _f32_key(chunk_v[pl.ds(o, 16)])
            gidx_v[pl.ds(o, 16)] = base + o + lane
        return 0

    lax.fori_loop(0, CVREGS // 4, xform, 0)
    t_loc, nc = _select_threshold(keys_v, gidx_v, CVREGS, hist_v, tot_v,
                                  ck_v, cg_v, K)
    nv4 = lax.shift_right_logical(nc + 63, jnp.int32(6))

    def extract(j, off):
        for u in range(4):
            o = j * 64 + u * 16
            kv = ck_v[pl.ds(o, 16)]
            gv = cg_v[pl.ds(o, 16)]
            m = kv >= t_loc
            mi = jnp.where(m, 1, 0).astype(jnp.int32)
            pos = off + plsc.cumsum(mi) - mi
            plsc.store_scatter(okey_v, [pos], kv, mask=m)
            plsc.store_scatter(ogid_v, [pos], gv, mask=m)
            off = off + jnp.sum(mi)
        return off

    lax.fori_loop(0, nv4, extract, jnp.int32(0))
    pltpu.sync_copy(okey_v.at[pl.ds(0, K)], ckeys_hbm.at[pl.ds(wid * K, K)])
    pltpu.sync_copy(ogid_v.at[pl.ds(0, K)], cgidx_hbm.at[pl.ds(wid * K, K)])


def _sc_merge_body(ckeys_hbm, cgidx_hbm, mem_hbm, out_hbm,
                   k2_v, g2_v, hist_v, tot_v, ck_v, cg_v,
                   skey_v, sgid_v, sidx_v, rows_v, sem):
    wid = lax.axis_index("s") * 2 + lax.axis_index("c")

    @pl.when(wid == 0)
    def _():
        pltpu.sync_copy(ckeys_hbm, k2_v)
        pltpu.sync_copy(cgidx_hbm, g2_v)
        t_g, nc = _select_threshold(k2_v, g2_v, MVREGS, hist_v, tot_v,
                                    ck_v, cg_v, K)
        nv4 = lax.shift_right_logical(nc + 63, jnp.int32(6))

        def cnt_gt(j, acc):
            for u in range(4):
                kv = ck_v[pl.ds(j * 64 + u * 16, 16)]
                acc = acc + jnp.where(kv > t_g, 1, 0).astype(jnp.int32)
            return acc

        n_gt = jnp.sum(lax.fori_loop(0, nv4, cnt_gt, jnp.zeros((16,), jnp.int32)))
        r = K - n_gt

        def extract(j, carry):
            off, eq_seen = carry
            for u in range(4):
                o = j * 64 + u * 16
                kv = ck_v[pl.ds(o, 16)]
                gv = cg_v[pl.ds(o, 16)]
                m_gt = kv > t_g
                m_eq = kv == t_g
                ei = jnp.where(m_eq, 1, 0).astype(jnp.int32)
                eq_rank = eq_seen + plsc.cumsum(ei) - ei
                m = m_gt | (m_eq & (eq_rank < r))
                mi = jnp.where(m, 1, 0).astype(jnp.int32)
                pos = off + plsc.cumsum(mi) - mi
                plsc.store_scatter(skey_v, [pos], kv, mask=m)
                plsc.store_scatter(sgid_v, [pos], gv, mask=m)
                off = off + jnp.sum(mi)
                eq_seen = eq_seen + jnp.sum(ei)
            return off, eq_seen

        lax.fori_loop(0, nv4, extract, (jnp.int32(0), jnp.int32(0)))

        kvs = [skey_v[pl.ds(v * 16, 16)] for v in range(4)]
        gvs = [sgid_v[pl.ds(v * 16, 16)] for v in range(4)]

        def rank_step(j, ranks):
            jv = jnp.broadcast_to(j, (16,))
            kj = plsc.load_gather(skey_v, [jv])
            gj = plsc.load_gather(sgid_v, [jv])
            out = []
            for v in range(4):
                pre = (kj > kvs[v]) | ((kj == kvs[v]) & (gj < gvs[v]))
                out.append(ranks[v] + jnp.where(pre, 1, 0).astype(jnp.int32))
            return tuple(out)

        ranks = lax.fori_loop(0, K, rank_step,
                              tuple(jnp.zeros((16,), jnp.int32) for _ in range(4)))
        for v in range(4):
            plsc.store_scatter(sidx_v, [ranks[v]], gvs[v])

        pltpu.async_copy(mem_hbm.at[sidx_v], rows_v, sem).wait()
        pltpu.sync_copy(rows_v, out_hbm)


NW1 = 16
CHUNK1 = M_PAD // NW1
CV1 = CHUNK1 // 16
NCAND1 = NW1 * K


def _sc_fused_body(sims_hbm, mem_hbm, out_hbm,
                   chunk_v, keys_v, gidx_v, hist_v, tot_v,
                   ck_v, cg_v, okey_v, ogid_v,
                   k2_v, g2_v, skey_v, sgid_v, sidx_v, rows_v,
                   shk_sh, shg_sh, sem):
    wid = lax.axis_index("s")
    base = wid * CHUNK1
    pltpu.sync_copy(sims_hbm.at[pl.ds(base, CHUNK1)], chunk_v)
    lane = lax.iota(jnp.int32, 16)

    def xform(j, _):
        for u in range(4):
            o = j * 64 + u * 16
            keys_v[pl.ds(o, 16)] = _f32_key(chunk_v[pl.ds(o, 16)])
            gidx_v[pl.ds(o, 16)] = base + o + lane
        return 0

    lax.fori_loop(0, CV1 // 4, xform, 0)
    t_loc, nc = _select_threshold(keys_v, gidx_v, CV1, hist_v, tot_v,
                                  ck_v, cg_v, K)
    nv4 = lax.shift_right_logical(nc + 63, jnp.int32(6))

    def extract(j, off):
        for u in range(4):
            o = j * 64 + u * 16
            kv = ck_v[pl.ds(o, 16)]
            gv = cg_v[pl.ds(o, 16)]
            m = kv >= t_loc
            mi = jnp.where(m, 1, 0).astype(jnp.int32)
            pos = off + plsc.cumsum(mi) - mi
            plsc.store_scatter(okey_v, [pos], kv, mask=m)
            plsc.store_scatter(ogid_v, [pos], gv, mask=m)
            off = off + jnp.sum(mi)
        return off

    lax.fori_loop(0, nv4, extract, jnp.int32(0))
    pltpu.sync_copy(okey_v.at[pl.ds(0, K)], shk_sh.at[pl.ds(wid * K, K)])
    pltpu.sync_copy(ogid_v.at[pl.ds(0, K)], shg_sh.at[pl.ds(wid * K, K)])
    plsc.subcore_barrier()

    @pl.when(wid == 0)
    def _():
        pltpu.sync_copy(shk_sh, k2_v)
        pltpu.sync_copy(shg_sh, g2_v)
        t_g, nc2 = _select_threshold(k2_v, g2_v, NCAND1 // 16, hist_v, tot_v,
                                     ck_v, cg_v, K)
        nv4b = lax.shift_right_logical(nc2 + 63, jnp.int32(6))

        def cnt_gt(j, acc):
            for u in range(4):
                kv = ck_v[pl.ds(j * 64 + u * 16, 16)]
                acc = acc + jnp.where(kv > t_g, 1, 0).astype(jnp.int32)
            return acc

        n_gt = jnp.sum(lax.fori_loop(0, nv4b, cnt_gt, jnp.zeros((16,), jnp.int32)))
        r = K - n_gt

        def extract2(j, carry):
            off, eq_seen = carry
            for u in range(4):
                o = j * 64 + u * 16
                kv = ck_v[pl.ds(o, 16)]
                gv = cg_v[pl.ds(o, 16)]
                m_gt = kv > t_g
                m_eq = kv == t_g
                ei = jnp.where(m_eq, 1, 0).astype(jnp.int32)
                eq_rank = eq_seen + plsc.cumsum(ei) - ei
                m = m_gt | (m_eq & (eq_rank < r))
                mi = jnp.where(m, 1, 0).astype(jnp.int32)
                pos = off + plsc.cumsum(mi) - mi
                plsc.store_scatter(skey_v, [pos], kv, mask=m)
                plsc.store_scatter(sgid_v, [pos], gv, mask=m)
                off = off + jnp.sum(mi)
                eq_seen = eq_seen + jnp.sum(ei)
            return off, eq_seen

        lax.fori_loop(0, nv4b, extract2, (jnp.int32(0), jnp.int32(0)))

        kvs = [skey_v[pl.ds(v * 16, 16)] for v in range(4)]
        gvs = [sgid_v[pl.ds(v * 16, 16)] for v in range(4)]

        def rank_step(j, ranks):
            jv = jnp.broadcast_to(j, (16,))
            kj = plsc.load_gather(skey_v, [jv])
            gj = plsc.load_gather(sgid_v, [jv])
            out = []
            for v in range(4):
                pre = (kj > kvs[v]) | ((kj == kvs[v]) & (gj < gvs[v]))
                out.append(ranks[v] + jnp.where(pre, 1, 0).astype(jnp.int32))
            return tuple(out)

        ranks = lax.fori_loop(0, K, rank_step,
                              tuple(jnp.zeros((16,), jnp.int32) for _ in range(4)))
        for v in range(4):
            plsc.store_scatter(sidx_v, [ranks[v]], gvs[v])

        pltpu.async_copy(mem_hbm.at[sidx_v], rows_v, sem).wait()
        pltpu.sync_copy(rows_v, out_hbm)


def _sc_topk_gather_fused(sims_flat, memory_tensor):
    mesh = plsc.VectorSubcoreMesh(core_axis_name="c", subcore_axis_name="s",
                                  num_cores=1)
    return pl.kernel(
        _sc_fused_body,
        out_type=jax.ShapeDtypeStruct((K, D), jnp.float32),
        mesh=mesh,
        compiler_params=pltpu.CompilerParams(needs_layout_passes=False),
        scratch_types=[
            pltpu.VMEM((CHUNK1,), jnp.float32),
            pltpu.VMEM((CHUNK1,), jnp.int32),
            pltpu.VMEM((CHUNK1,), jnp.int32),
            pltpu.VMEM((4096,), jnp.int32),
            pltpu.VMEM((256,), jnp.int32),
            pltpu.VMEM((CHUNK1 + 64,), jnp.int32),
            pltpu.VMEM((CHUNK1 + 64,), jnp.int32),
            pltpu.VMEM((CHUNK1 + 64,), jnp.int32),
            pltpu.VMEM((CHUNK1 + 64,), jnp.int32),
            pltpu.VMEM((NCAND1,), jnp.int32),
            pltpu.VMEM((NCAND1,), jnp.int32),
            pltpu.VMEM((K,), jnp.int32),
            pltpu.VMEM((K,), jnp.int32),
            pltpu.VMEM((K,), jnp.int32),
            pltpu.VMEM((K, D), jnp.float32),
            pltpu.VMEM_SHARED((NCAND1,), jnp.int32),
            pltpu.VMEM_SHARED((NCAND1,), jnp.int32),
            pltpu.SemaphoreType.DMA,
        ],
    )(sims_flat, memory_tensor)


def _sc_topk_gather(sims_flat, memory_tensor):
    mesh = plsc.VectorSubcoreMesh(core_axis_name="c", subcore_axis_name="s")
    ckeys, cgidx = pl.kernel(
        _sc_local_body,
        out_type=(jax.ShapeDtypeStruct((NCAND,), jnp.int32),
                  jax.ShapeDtypeStruct((NCAND,), jnp.int32)),
        mesh=mesh,
        compiler_params=pltpu.CompilerParams(needs_layout_passes=False),
        scratch_types=[
            pltpu.VMEM((CHUNK,), jnp.float32),
            pltpu.VMEM((CHUNK,), jnp.int32),
            pltpu.VMEM((CHUNK,), jnp.int32),
            pltpu.VMEM((4096,), jnp.int32),
            pltpu.VMEM((256,), jnp.int32),
            pltpu.VMEM((CHUNK + 64,), jnp.int32),
            pltpu.VMEM((CHUNK + 64,), jnp.int32),
            pltpu.VMEM((CHUNK + 64,), jnp.int32),
            pltpu.VMEM((CHUNK + 64,), jnp.int32),
        ],
    )(sims_flat)
    return pl.kernel(
        _sc_merge_body,
        out_type=jax.ShapeDtypeStruct((K, D), jnp.float32),
        mesh=mesh,
        compiler_params=pltpu.CompilerParams(needs_layout_passes=False),
        scratch_types=[
            pltpu.VMEM((NCAND,), jnp.int32),
            pltpu.VMEM((NCAND,), jnp.int32),
            pltpu.VMEM((4096,), jnp.int32),
            pltpu.VMEM((256,), jnp.int32),
            pltpu.VMEM((NCAND + 64,), jnp.int32),
            pltpu.VMEM((NCAND + 64,), jnp.int32),
            pltpu.VMEM((K,), jnp.int32),
            pltpu.VMEM((K,), jnp.int32),
            pltpu.VMEM((K,), jnp.int32),
            pltpu.VMEM((K, D), jnp.float32),
            pltpu.SemaphoreType.DMA,
        ],
    )(ckeys, cgidx, memory_tensor)


def _topk_body(sims_ref, idx_ref):
    sims = sims_ref[...]
    r = jax.lax.broadcasted_iota(jnp.int32, (ROWS, 128), 0)
    c = jax.lax.broadcasted_iota(jnp.int32, (ROWS, 128), 1)
    flat = r * 128 + c

    def body(k, s):
        m = jnp.max(s)
        i = jnp.min(jnp.where(s == m, flat, jnp.int32(2**30)))
        idx_ref[k] = i
        return jnp.where(flat == i, NEG, s)

    jax.lax.fori_loop(0, K, body, sims)


def _topk_idx(sims2d):
    return pl.pallas_call(
        _topk_body,
        out_specs=pl.BlockSpec(memory_space=pltpu.SMEM),
        out_shape=jax.ShapeDtypeStruct((K,), jnp.int32),
    )(sims2d)


def _gather_body(idx_ref, mem_ref, out_ref):
    out_ref[...] = mem_ref[...]


def _gather_rows(idx, memory_tensor):
    grid_spec = pltpu.PrefetchScalarGridSpec(
        num_scalar_prefetch=1,
        grid=(K,),
        in_specs=[pl.BlockSpec((1, 1, D), lambda i, idx_ref: (idx_ref[i], 0, 0))],
        out_specs=pl.BlockSpec((1, 1, D), lambda i, idx_ref: (i, 0, 0)),
    )
    out = pl.pallas_call(
        _gather_body,
        grid_spec=grid_spec,
        out_shape=jax.ShapeDtypeStruct((K, 1, D), jnp.float32),
    )(idx, memory_tensor.reshape(M, 1, D))
    return out.reshape(K, D)


def _layer_norm_in(x, w, b):
    m = jnp.mean(x, axis=-1, keepdims=True)
    d = x - m
    v = jnp.mean(d * d, axis=-1, keepdims=True)
    return d * (1.0 / jnp.sqrt(v + 1e-5)) * w + b


def _tfm_body(x0_ref, wi_ref, bi_ref, wo_ref, bo_ref, w1_ref, b1_ref,
              w2_ref, b2_ref, l1w_ref, l1b_ref, l2w_ref, l2b_ref, out_ref):
    x = x0_ref[...]
    inv_sqrt = 1.0 / (DH ** 0.5)
    SH = K * H
    r6 = jax.lax.broadcasted_iota(jnp.int32, (SH, SH), 0) // K
    c6 = jax.lax.broadcasted_iota(jnp.int32, (SH, SH), 1) // K
    blk = r6 == c6
    for l in range(L):
        qkv = jnp.dot(x, wi_ref[l], preferred_element_type=jnp.float32) + bi_ref[l]
        qh = jnp.concatenate([qkv[:, h * DH:(h + 1) * DH] for h in range(H)], axis=0)
        kh = jnp.concatenate([qkv[:, D + h * DH:D + (h + 1) * DH] for h in range(H)], axis=0)
        vh = jnp.concatenate([qkv[:, 2 * D + h * DH:2 * D + (h + 1) * DH] for h in range(H)], axis=0)
        scores = jax.lax.dot_general(
            qh, kh, (((1,), (1,)), ((), ())),
            preferred_element_type=jnp.float32) * inv_sqrt
        scores = jnp.where(blk, scores, NEG)
        mx = jnp.max(scores, axis=-1, keepdims=True)
        e = jnp.exp(scores - mx)
        attn = e / jnp.sum(e, axis=-1, keepdims=True)
        oh = jnp.dot(attn, vh, preferred_element_type=jnp.float32)
        a = jnp.concatenate([oh[h * K:(h + 1) * K, :] for h in range(H)], axis=1)
        a = jnp.dot(a, wo_ref[l], preferred_element_type=jnp.float32) + bo_ref[l]
        x = _layer_norm_in(x + a, l1w_ref[l], l1b_ref[l])
        ff = jnp.maximum(jnp.dot(x, w1_ref[l], preferred_element_type=jnp.float32) + b1_ref[l], 0.0)
        ff = jnp.dot(ff, w2_ref[l], preferred_element_type=jnp.float32) + b2_ref[l]
        x = _layer_norm_in(x + ff, l2w_ref[l], l2b_ref[l])
    out_ref[...] = x


def _transformer(x0, wi_t, bi, wo_t, bo, w1_t, b1, w2_t, b2, l1w, l1b, l2w, l2b):
    return pl.pallas_call(
        _tfm_body,
        out_shape=jax.ShapeDtypeStruct((K, D), jnp.float32),
    )(x0, wi_t, bi, wo_t, bo, w1_t, b1, w2_t, b2, l1w, l1b, l2w, l2b)


def kernel(current_context, memory_tensor, W_in, b_in, W_out, b_out,
           W1, b1, W2, b2, ln1_w, ln1_b, ln2_w, ln2_b, top_k):
    ctx_row = current_context.reshape(1, D)
    sims = _compute_sims(memory_tensor, ctx_row)
    x0 = _sc_topk_gather_fused(sims, memory_tensor)

    wi_t = jnp.transpose(W_in, (0, 2, 1))
    wo_t = jnp.transpose(W_out, (0, 2, 1))
    w1_t = jnp.transpose(W1, (0, 2, 1))
    w2_t = jnp.transpose(W2, (0, 2, 1))
    bi = b_in.reshape(L, 1, 3 * D)
    bo = b_out.reshape(L, 1, D)
    b1r = b1.reshape(L, 1, FF)
    b2r = b2.reshape(L, 1, D)
    l1w = ln1_w.reshape(L, 1, D)
    l1b = ln1_b.reshape(L, 1, D)
    l2w = ln2_w.reshape(L, 1, D)
    l2b = ln2_b.reshape(L, 1, D)
    return _transformer(x0, wi_t, bi, wo_t, bo, w1_t, b1r, w2_t, b2r,
                        l1w, l1b, l2w, l2b)


# speedup vs baseline: 1.7665x; 1.0034x over previous
import functools

import jax
import jax.numpy as jnp
from jax import lax
from jax.experimental import pallas as pl
from jax.experimental.pallas import tpu as pltpu
from jax.experimental.pallas import tpu_sc as plsc

D = 128
M = 100000
K = 64
L = 4
H = 8
DH = 16
FF = 512

MV_GRID = 7
MV_BLOCK = 14336
M_PAD = MV_GRID * MV_BLOCK
ROWS = M_PAD // 128

NEG = float("-inf")

NW = 32
CHUNK = M_PAD // NW
CVREGS = CHUNK // 16
NCAND = NW * K
MVREGS = NCAND // 16


def _sims_body(mem_ref, ctx_ref, out_ref):
    i = pl.program_id(0)
    s = jax.lax.dot_general(ctx_ref[...], mem_ref[...], (((1,), (1,)), ((), ())),
                            preferred_element_type=jnp.float32)
    col = jax.lax.broadcasted_iota(jnp.int32, (1, MV_BLOCK), 1) + i * MV_BLOCK
    out_ref[...] = jnp.where(col < M, s, NEG).reshape(MV_BLOCK)


def _compute_sims(memory_tensor, ctx_row):
    return pl.pallas_call(
        _sims_body,
        grid=(MV_GRID,),
        in_specs=[
            pl.BlockSpec((MV_BLOCK, D), lambda i: (i, 0)),
            pl.BlockSpec((1, D), lambda i: (0, 0)),
        ],
        out_specs=pl.BlockSpec((MV_BLOCK,), lambda i: (i,)),
        out_shape=jax.ShapeDtypeStruct((M_PAD,), jnp.float32),
    )(memory_tensor, ctx_row)


_SIGN_INT = -(2 ** 31)


def _f32_key(v):
    u = plsc.bitcast(v, jnp.int32)
    return jnp.where(u < 0, jnp.int32(-1) - (u + jnp.int32(_SIGN_INT)), u)


def _select_threshold(keys_ref, gidx_ref, nvregs, hist_v, tot_v, ck_v, cg_v, k):
    sign = jnp.int32(_SIGN_INT)
    lane = lax.iota(jnp.int32, 16)
    ones = jnp.ones((16,), jnp.int32)
    zeros = jnp.zeros((16,), jnp.int32)

    def zh(j, _):
        for u in range(8):
            hist_v[pl.ds(j * 128 + u * 16, 16)] = zeros
        return 0

    lax.fori_loop(0, 32, zh, 0)

    def hpass(j, _):
        for u in range(4):
            kv = keys_ref[pl.ds(j * 64 + u * 16, 16)]
            b = lax.shift_right_logical(kv + sign, jnp.int32(24))
            plsc.addupdate_scatter(hist_v, [lane * 256 + b], ones)
        return 0

    lax.fori_loop(0, nvregs // 4, hpass, 0)

    for g in range(16):
        acc = zeros
        for l in range(16):
            acc = acc + hist_v[pl.ds(l * 256 + g * 16, 16)]
        tot_v[pl.ds(g * 16, 16)] = acc

    run = jnp.int32(0)
    b_star = jnp.int32(0)
    found = jnp.int32(0)
    for g in range(15, -1, -1):
        tv = tot_v[pl.ds(g * 16, 16)]
        rv = lax.rev(tv, (0,))
        cs = plsc.cumsum(rv)
        m = (run + cs) >= k
        mi = jnp.where(m, 1, 0).astype(jnp.int32)
        any_m = jnp.sum(mi)
        mf = m & (plsc.cumsum(mi) == 1)
        i_star = jnp.sum(jnp.where(mf, lane, 0))
        b_g = g * 16 + 15 - i_star
        hit = (found == 0) & (any_m > 0)
        b_star = jnp.where(hit, b_g, b_star)
        found = jnp.where(hit, 1, found)
        run = run + jnp.sum(tv)

    ts8 = b_star * jnp.int32(1 << 24) + sign

    def cpass(j, off):
        for u in range(4):
            kv = keys_ref[pl.ds(j * 64 + u * 16, 16)]
            gv = gidx_ref[pl.ds(j * 64 + u * 16, 16)]
            m = kv >= ts8
            mi = jnp.where(m, 1, 0).astype(jnp.int32)
            pos = off + plsc.cumsum(mi) - mi
            plsc.store_scatter(ck_v, [pos], kv, mask=m)
            plsc.store_scatter(cg_v, [pos], gv, mask=m)
            off = off + jnp.sum(mi)
        return off

    nc = lax.fori_loop(0, nvregs // 4, cpass, jnp.int32(0))

    pad = jnp.broadcast_to(sign, (16,))
    for u in range(4):
        ck_v[pl.ds(nc + u * 16, 16)] = pad
    nv4 = lax.shift_right_logical(nc + 63, jnp.int32(6))

    def one_bit(_, carry):
        p, bit = carry
        t = p + bit
        ts = t + sign

        def cnt(j, acc):
            for u in range(4):
                kv = ck_v[pl.ds(j * 64 + u * 16, 16)]
                acc = acc + jnp.where(kv >= ts, 1, 0).astype(jnp.int32)
            return acc

        c = jnp.sum(lax.fori_loop(0, nv4, cnt, zeros))
        p = jnp.where(c >= k, t, p)
        return p, lax.shift_right_logical(bit, jnp.int32(1))

    p0 = b_star * jnp.int32(1 << 24)
    p, _ = lax.fori_loop(0, 24, one_bit, (p0, jnp.int32(1 << 23)))
    return p + sign, nc


def _sc_local_body(sims_hbm, ckeys_hbm, cgidx_hbm,
                   chunk_v, keys_v, gidx_v, hist_v, tot_v,
                   ck_v, cg_v, okey_v, ogid_v):
    wid = lax.axis_index("s") * 2 + lax.axis_index("c")
    base = wid * CHUNK
    pltpu.sync_copy(sims_hbm.at[pl.ds(base, CHUNK)], chunk_v)
    lane = lax.iota(jnp.int32, 16)

    def xform(j, _):
        for u in range(4):
            o = j * 64 + u * 16
            keys_v[pl.ds(o, 16)] = _f32_key(chunk_v[pl.ds(o, 16)])
            gidx_v[pl.ds(o, 16)] = base + o + lane
        return 0

    lax.fori_loop(0, CVREGS // 4, xform, 0)
    t_loc, nc = _select_threshold(keys_v, gidx_v, CVREGS, hist_v, tot_v,
                                  ck_v, cg_v, K)
    nv4 = lax.shift_right_logical(nc + 63, jnp.int32(6))

    def extract(j, off):
        for u in range(4):
            o = j * 64 + u * 16
            kv = ck_v[pl.ds(o, 16)]
            gv = cg_v[pl.ds(o, 16)]
            m = kv >= t_loc
            mi = jnp.where(m, 1, 0).astype(jnp.int32)
            pos = off + plsc.cumsum(mi) - mi
            plsc.store_scatter(okey_v, [pos], kv, mask=m)
            plsc.store_scatter(ogid_v, [pos], gv, mask=m)
            off = off + jnp.sum(mi)
        return off

    lax.fori_loop(0, nv4, extract, jnp.int32(0))
    pltpu.sync_copy(okey_v.at[pl.ds(0, K)], ckeys_hbm.at[pl.ds(wid * K, K)])
    pltpu.sync_copy(ogid_v.at[pl.ds(0, K)], cgidx_hbm.at[pl.ds(wid * K, K)])


def _sc_merge_body(ckeys_hbm, cgidx_hbm, mem_hbm, out_hbm,
                   k2_v, g2_v, hist_v, tot_v, ck_v, cg_v,
                   skey_v, sgid_v, sidx_v, rows_v, sem):
    wid = lax.axis_index("s") * 2 + lax.axis_index("c")

    @pl.when(wid == 0)
    def _():
        pltpu.sync_copy(ckeys_hbm, k2_v)
        pltpu.sync_copy(cgidx_hbm, g2_v)
        t_g, nc = _select_threshold(k2_v, g2_v, MVREGS, hist_v, tot_v,
                                    ck_v, cg_v, K)
        nv4 = lax.shift_right_logical(nc + 63, jnp.int32(6))

        def cnt_gt(j, acc):
            for u in range(4):
                kv = ck_v[pl.ds(j * 64 + u * 16, 16)]
                acc = acc + jnp.where(kv > t_g, 1, 0).astype(jnp.int32)
            return acc

        n_gt = jnp.sum(lax.fori_loop(0, nv4, cnt_gt, jnp.zeros((16,), jnp.int32)))
        r = K - n_gt

        def extract(j, carry):
            off, eq_seen = carry
            for u in range(4):
                o = j * 64 + u * 16
                kv = ck_v[pl.ds(o, 16)]
                gv = cg_v[pl.ds(o, 16)]
                m_gt = kv > t_g
                m_eq = kv == t_g
                ei = jnp.where(m_eq, 1, 0).astype(jnp.int32)
                eq_rank = eq_seen + plsc.cumsum(ei) - ei
                m = m_gt | (m_eq & (eq_rank < r))
                mi = jnp.where(m, 1, 0).astype(jnp.int32)
                pos = off + plsc.cumsum(mi) - mi
                plsc.store_scatter(skey_v, [pos], kv, mask=m)
                plsc.store_scatter(sgid_v, [pos], gv, mask=m)
                off = off + jnp.sum(mi)
                eq_seen = eq_seen + jnp.sum(ei)
            return off, eq_seen

        lax.fori_loop(0, nv4, extract, (jnp.int32(0), jnp.int32(0)))

        kvs = [skey_v[pl.ds(v * 16, 16)] for v in range(4)]
        gvs = [sgid_v[pl.ds(v * 16, 16)] for v in range(4)]

        def rank_step(j, ranks):
            jv = jnp.broadcast_to(j, (16,))
            kj = plsc.load_gather(skey_v, [jv])
            gj = plsc.load_gather(sgid_v, [jv])
            out = []
            for v in range(4):
                pre = (kj > kvs[v]) | ((kj == kvs[v]) & (gj < gvs[v]))
                out.append(ranks[v] + jnp.where(pre, 1, 0).astype(jnp.int32))
            return tuple(out)

        ranks = lax.fori_loop(0, K, rank_step,
                              tuple(jnp.zeros((16,), jnp.int32) for _ in range(4)))
        for v in range(4):
            plsc.store_scatter(sidx_v, [ranks[v]], gvs[v])

        pltpu.async_copy(mem_hbm.at[sidx_v], rows_v, sem).wait()
        pltpu.sync_copy(rows_v, out_hbm)


NW1 = 16
CHUNK1 = M_PAD // NW1
CV1 = CHUNK1 // 16
NCAND1 = NW1 * K


def _sc_fused_body(sims_hbm, mem_hbm, out_hbm,
                   chunk_v, keys_v, gidx_v, hist_v, tot_v,
                   ck_v, cg_v, okey_v, ogid_v,
                   k2_v, g2_v, skey_v, sgid_v, sidx_v, rows_v,
                   shk_sh, shg_sh, sem):
    wid = lax.axis_index("s")
    base = wid * CHUNK1
    pltpu.sync_copy(sims_hbm.at[pl.ds(base, CHUNK1)], chunk_v)
    lane = lax.iota(jnp.int32, 16)

    def xform(j, _):
        for u in range(4):
            o = j * 64 + u * 16
            keys_v[pl.ds(o, 16)] = _f32_key(chunk_v[pl.ds(o, 16)])
            gidx_v[pl.ds(o, 16)] = base + o + lane
        return 0

    lax.fori_loop(0, CV1 // 4, xform, 0)
    t_loc, nc = _select_threshold(keys_v, gidx_v, CV1, hist_v, tot_v,
                                  ck_v, cg_v, K)
    nv4 = lax.shift_right_logical(nc + 63, jnp.int32(6))

    def extract(j, off):
        for u in range(4):
            o = j * 64 + u * 16
            kv = ck_v[pl.ds(o, 16)]
            gv = cg_v[pl.ds(o, 16)]
            m = kv >= t_loc
            mi = jnp.where(m, 1, 0).astype(jnp.int32)
            pos = off + plsc.cumsum(mi) - mi
            plsc.store_scatter(okey_v, [pos], kv, mask=m)
            plsc.store_scatter(ogid_v, [pos], gv, mask=m)
            off = off + jnp.sum(mi)
        return off

    lax.fori_loop(0, nv4, extract, jnp.int32(0))
    pltpu.sync_copy(okey_v.at[pl.ds(0, K)], shk_sh.at[pl.ds(wid * K, K)])
    pltpu.sync_copy(ogid_v.at[pl.ds(0, K)], shg_sh.at[pl.ds(wid * K, K)])
    plsc.subcore_barrier()

    @pl.when(wid == 0)
    def _():
        pltpu.sync_copy(shk_sh, k2_v)
        pltpu.sync_copy(shg_sh, g2_v)
        t_g, nc2 = _select_threshold(k2_v, g2_v, NCAND1 // 16, hist_v, tot_v,
                                     ck_v, cg_v, K)
        nv4b = lax.shift_right_logical(nc2 + 63, jnp.int32(6))

        def cnt_gt(j, acc):
            for u in range(4):
                kv = ck_v[pl.ds(j * 64 + u * 16, 16)]
                acc = acc + jnp.where(kv > t_g, 1, 0).astype(jnp.int32)
            return acc

        n_gt = jnp.sum(lax.fori_loop(0, nv4b, cnt_gt, jnp.zeros((16,), jnp.int32)))
        r = K - n_gt

        def extract2(j, carry):
            off, eq_seen = carry
            for u in range(4):
                o = j * 64 + u * 16
                kv = ck_v[pl.ds(o, 16)]
                gv = cg_v[pl.ds(o, 16)]
                m_gt = kv > t_g
                m_eq = kv == t_g
                ei = jnp.where(m_eq, 1, 0).astype(jnp.int32)
                eq_rank = eq_seen + plsc.cumsum(ei) - ei
                m = m_gt | (m_eq & (eq_rank < r))
                mi = jnp.where(m, 1, 0).astype(jnp.int32)
                pos = off + plsc.cumsum(mi) - mi
                plsc.store_scatter(skey_v, [pos], kv, mask=m)
                plsc.store_scatter(sgid_v, [pos], gv, mask=m)
                off = off + jnp.sum(mi)
                eq_seen = eq_seen + jnp.sum(ei)
            return off, eq_seen

        lax.fori_loop(0, nv4b, extract2, (jnp.int32(0), jnp.int32(0)))

        kvs = [skey_v[pl.ds(v * 16, 16)] for v in range(4)]
        gvs = [sgid_v[pl.ds(v * 16, 16)] for v in range(4)]

        def rank_step(j, ranks):
            jv = jnp.broadcast_to(j, (16,))
            kj = plsc.load_gather(skey_v, [jv])
            gj = plsc.load_gather(sgid_v, [jv])
            out = []
            for v in range(4):
                pre = (kj > kvs[v]) | ((kj == kvs[v]) & (gj < gvs[v]))
                out.append(ranks[v] + jnp.where(pre, 1, 0).astype(jnp.int32))
            return tuple(out)

        ranks = lax.fori_loop(0, K, rank_step,
                              tuple(jnp.zeros((16,), jnp.int32) for _ in range(4)))
        for v in range(4):
            plsc.store_scatter(sidx_v, [ranks[v]], gvs[v])

        pltpu.async_copy(mem_hbm.at[sidx_v], rows_v, sem).wait()
        pltpu.sync_copy(rows_v, out_hbm)


def _sc_topk_gather_fused(sims_flat, memory_tensor):
    mesh = plsc.VectorSubcoreMesh(core_axis_name="c", subcore_axis_name="s",
                                  num_cores=1)
    return pl.kernel(
        _sc_fused_body,
        out_type=jax.ShapeDtypeStruct((K, D), jnp.float32),
        mesh=mesh,
        compiler_params=pltpu.CompilerParams(needs_layout_passes=False),
        scratch_types=[
            pltpu.VMEM((CHUNK1,), jnp.float32),
            pltpu.VMEM((CHUNK1,), jnp.int32),
            pltpu.VMEM((CHUNK1,), jnp.int32),
            pltpu.VMEM((4096,), jnp.int32),
            pltpu.VMEM((256,), jnp.int32),
            pltpu.VMEM((CHUNK1 + 64,), jnp.int32),
            pltpu.VMEM((CHUNK1 + 64,), jnp.int32),
            pltpu.VMEM((CHUNK1 + 64,), jnp.int32),
            pltpu.VMEM((CHUNK1 + 64,), jnp.int32),
            pltpu.VMEM((NCAND1,), jnp.int32),
            pltpu.VMEM((NCAND1,), jnp.int32),
            pltpu.VMEM((K,), jnp.int32),
            pltpu.VMEM((K,), jnp.int32),
            pltpu.VMEM((K,), jnp.int32),
            pltpu.VMEM((K, D), jnp.float32),
            pltpu.VMEM_SHARED((NCAND1,), jnp.int32),
            pltpu.VMEM_SHARED((NCAND1,), jnp.int32),
            pltpu.SemaphoreType.DMA,
        ],
    )(sims_flat, memory_tensor)


def _sc_topk_gather(sims_flat, memory_tensor):
    mesh = plsc.VectorSubcoreMesh(core_axis_name="c", subcore_axis_name="s")
    ckeys, cgidx = pl.kernel(
        _sc_local_body,
        out_type=(jax.ShapeDtypeStruct((NCAND,), jnp.int32),
                  jax.ShapeDtypeStruct((NCAND,), jnp.int32)),
        mesh=mesh,
        compiler_params=pltpu.CompilerParams(needs_layout_passes=False),
        scratch_types=[
            pltpu.VMEM((CHUNK,), jnp.float32),
            pltpu.VMEM((CHUNK,), jnp.int32),
            pltpu.VMEM((CHUNK,), jnp.int32),
            pltpu.VMEM((4096,), jnp.int32),
            pltpu.VMEM((256,), jnp.int32),
            pltpu.VMEM((CHUNK + 64,), jnp.int32),
            pltpu.VMEM((CHUNK + 64,), jnp.int32),
            pltpu.VMEM((CHUNK + 64,), jnp.int32),
            pltpu.VMEM((CHUNK + 64,), jnp.int32),
        ],
    )(sims_flat)
    return pl.kernel(
        _sc_merge_body,
        out_type=jax.ShapeDtypeStruct((K, D), jnp.float32),
        mesh=mesh,
        compiler_params=pltpu.CompilerParams(needs_layout_passes=False),
        scratch_types=[
            pltpu.VMEM((NCAND,), jnp.int32),
            pltpu.VMEM((NCAND,), jnp.int32),
            pltpu.VMEM((4096,), jnp.int32),
            pltpu.VMEM((256,), jnp.int32),
            pltpu.VMEM((NCAND + 64,), jnp.int32),
            pltpu.VMEM((NCAND + 64,), jnp.int32),
            pltpu.VMEM((K,), jnp.int32),
            pltpu.VMEM((K,), jnp.int32),
            pltpu.VMEM((K,), jnp.int32),
            pltpu.VMEM((K, D), jnp.float32),
            pltpu.SemaphoreType.DMA,
        ],
    )(ckeys, cgidx, memory_tensor)


def _topk_body(sims_ref, idx_ref):
    sims = sims_ref[...]
    r = jax.lax.broadcasted_iota(jnp.int32, (ROWS, 128), 0)
    c = jax.lax.broadcasted_iota(jnp.int32, (ROWS, 128), 1)
    flat = r * 128 + c

    def body(k, s):
        m = jnp.max(s)
        i = jnp.min(jnp.where(s == m, flat, jnp.int32(2**30)))
        idx_ref[k] = i
        return jnp.where(flat == i, NEG, s)

    jax.lax.fori_loop(0, K, body, sims)


def _topk_idx(sims2d):
    return pl.pallas_call(
        _topk_body,
        out_specs=pl.BlockSpec(memory_space=pltpu.SMEM),
        out_shape=jax.ShapeDtypeStruct((K,), jnp.int32),
    )(sims2d)


def _gather_body(idx_ref, mem_ref, out_ref):
    out_ref[...] = mem_ref[...]


def _gather_rows(idx, memory_tensor):
    grid_spec = pltpu.PrefetchScalarGridSpec(
        num_scalar_prefetch=1,
        grid=(K,),
        in_specs=[pl.BlockSpec((1, 1, D), lambda i, idx_ref: (idx_ref[i], 0, 0))],
        out_specs=pl.BlockSpec((1, 1, D), lambda i, idx_ref: (i, 0, 0)),
    )
    out = pl.pallas_call(
        _gather_body,
        grid_spec=grid_spec,
        out_shape=jax.ShapeDtypeStruct((K, 1, D), jnp.float32),
    )(idx, memory_tensor.reshape(M, 1, D))
    return out.reshape(K, D)


def _layer_norm_in(x, w, b):
    m = jnp.mean(x, axis=-1, keepdims=True)
    d = x - m
    v = jnp.mean(d * d, axis=-1, keepdims=True)
    return d * (1.0 / jnp.sqrt(v + 1e-5)) * w + b


def _tfm_body(x0_ref, wi_ref, bi_ref, wo_ref, bo_ref, w1_ref, b1_ref,
              w2_ref, b2_ref, l1w_ref, l1b_ref, l2w_ref, l2b_ref, out_ref):
    x = x0_ref[...]
    inv_sqrt = 1.0 / (DH ** 0.5)
    SH = K * H
    r6 = jax.lax.broadcasted_iota(jnp.int32, (SH, SH), 0) // K
    c6 = jax.lax.broadcasted_iota(jnp.int32, (SH, SH), 1) // K
    blk = r6 == c6
    for l in range(L):
        qkv = lax.dot_general(x, wi_ref[l], (((1,), (1,)), ((), ())),
                              preferred_element_type=jnp.float32) + bi_ref[l]
        qh = jnp.concatenate([qkv[:, h * DH:(h + 1) * DH] for h in range(H)], axis=0)
        kh = jnp.concatenate([qkv[:, D + h * DH:D + (h + 1) * DH] for h in range(H)], axis=0)
        vh = jnp.concatenate([qkv[:, 2 * D + h * DH:2 * D + (h + 1) * DH] for h in range(H)], axis=0)
        scores = jax.lax.dot_general(
            qh, kh, (((1,), (1,)), ((), ())),
            preferred_element_type=jnp.float32) * inv_sqrt
        scores = jnp.where(blk, scores, NEG)
        mx = jnp.max(scores, axis=-1, keepdims=True)
        e = jnp.exp(scores - mx)
        attn = e / jnp.sum(e, axis=-1, keepdims=True)
        oh = jnp.dot(attn, vh, preferred_element_type=jnp.float32)
        a = jnp.concatenate([oh[h * K:(h + 1) * K, :] for h in range(H)], axis=1)
        a = lax.dot_general(a, wo_ref[l], (((1,), (1,)), ((), ())),
                            preferred_element_type=jnp.float32) + bo_ref[l]
        x = _layer_norm_in(x + a, l1w_ref[l], l1b_ref[l])
        ff = jnp.maximum(lax.dot_general(x, w1_ref[l], (((1,), (1,)), ((), ())),
                                         preferred_element_type=jnp.float32) + b1_ref[l], 0.0)
        ff = lax.dot_general(ff, w2_ref[l], (((1,), (1,)), ((), ())),
                             preferred_element_type=jnp.float32) + b2_ref[l]
        x = _layer_norm_in(x + ff, l2w_ref[l], l2b_ref[l])
    out_ref[...] = x


def _transformer(x0, wi_t, bi, wo_t, bo, w1_t, b1, w2_t, b2, l1w, l1b, l2w, l2b):
    return pl.pallas_call(
        _tfm_body,
        out_shape=jax.ShapeDtypeStruct((K, D), jnp.float32),
    )(x0, wi_t, bi, wo_t, bo, w1_t, b1, w2_t, b2, l1w, l1b, l2w, l2b)


def kernel(current_context, memory_tensor, W_in, b_in, W_out, b_out,
           W1, b1, W2, b2, ln1_w, ln1_b, ln2_w, ln2_b, top_k):
    ctx_row = current_context.reshape(1, D)
    sims = _compute_sims(memory_tensor, ctx_row)
    x0 = _sc_topk_gather_fused(sims, memory_tensor)

    wi_t = W_in
    wo_t = W_out
    w1_t = W1
    w2_t = W2
    bi = b_in.reshape(L, 1, 3 * D)
    bo = b_out.reshape(L, 1, D)
    b1r = b1.reshape(L, 1, FF)
    b2r = b2.reshape(L, 1, D)
    l1w = ln1_w.reshape(L, 1, D)
    l1b = ln1_b.reshape(L, 1, D)
    l2w = ln2_w.reshape(L, 1, D)
    l2b = ln2_b.reshape(L, 1, D)
    return _transformer(x0, wi_t, bi, wo_t, bo, w1_t, b1r, w2_t, b2r,
                        l1w, l1b, l2w, l2b)


# speedup vs baseline: 1.8049x; 1.0217x over previous
import functools

import jax
import jax.numpy as jnp
from jax import lax
from jax.experimental import pallas as pl
from jax.experimental.pallas import tpu as pltpu
from jax.experimental.pallas import tpu_sc as plsc

D = 128
M = 100000
K = 64
L = 4
H = 8
DH = 16
FF = 512

MV_GRID = 7
MV_BLOCK = 14336
M_PAD = MV_GRID * MV_BLOCK
ROWS = M_PAD // 128

NEG = float("-inf")

NW = 32
CHUNK = M_PAD // NW
CVREGS = CHUNK // 16
NCAND = NW * K
MVREGS = NCAND // 16


def _sims_body(mem_ref, ctx_ref, out_ref):
    i = pl.program_id(0)
    s = jax.lax.dot_general(ctx_ref[...], mem_ref[...], (((1,), (1,)), ((), ())),
                            preferred_element_type=jnp.float32)
    col = jax.lax.broadcasted_iota(jnp.int32, (1, MV_BLOCK), 1) + i * MV_BLOCK
    out_ref[...] = jnp.where(col < M, s, NEG).reshape(MV_BLOCK)


def _compute_sims(memory_tensor, ctx_row):
    return pl.pallas_call(
        _sims_body,
        grid=(MV_GRID,),
        in_specs=[
            pl.BlockSpec((MV_BLOCK, D), lambda i: (i, 0)),
            pl.BlockSpec((1, D), lambda i: (0, 0)),
        ],
        out_specs=pl.BlockSpec((MV_BLOCK,), lambda i: (i,)),
        out_shape=jax.ShapeDtypeStruct((M_PAD,), jnp.float32),
    )(memory_tensor, ctx_row)


_SIGN_INT = -(2 ** 31)


def _f32_key(v):
    u = plsc.bitcast(v, jnp.int32)
    return jnp.where(u < 0, jnp.int32(-1) - (u + jnp.int32(_SIGN_INT)), u)


def _select_threshold(keys_ref, nvregs, hist_v, tot_v, ck_v, cg_v, k,
                      chunk_ref=None, gidx_ref=None, gbase=None):
    sign = jnp.int32(_SIGN_INT)
    lane = lax.iota(jnp.int32, 16)
    ones = jnp.ones((16,), jnp.int32)
    zeros = jnp.zeros((16,), jnp.int32)

    def zh(j, _):
        for u in range(8):
            hist_v[pl.ds(j * 128 + u * 16, 16)] = zeros
        return 0

    lax.fori_loop(0, 32, zh, 0)

    def hpass(j, _):
        for u in range(4):
            o = j * 64 + u * 16
            if chunk_ref is not None:
                kv = _f32_key(chunk_ref[pl.ds(o, 16)])
                keys_ref[pl.ds(o, 16)] = kv
            else:
                kv = keys_ref[pl.ds(o, 16)]
            b = lax.shift_right_logical(kv + sign, jnp.int32(24))
            plsc.addupdate_scatter(hist_v, [lane * 256 + b], ones)
        return 0

    lax.fori_loop(0, nvregs // 4, hpass, 0)

    for g in range(16):
        acc = zeros
        for l in range(16):
            acc = acc + hist_v[pl.ds(l * 256 + g * 16, 16)]
        tot_v[pl.ds(g * 16, 16)] = acc

    run = jnp.int32(0)
    b_star = jnp.int32(0)
    found = jnp.int32(0)
    for g in range(15, -1, -1):
        tv = tot_v[pl.ds(g * 16, 16)]
        rv = lax.rev(tv, (0,))
        cs = plsc.cumsum(rv)
        m = (run + cs) >= k
        mi = jnp.where(m, 1, 0).astype(jnp.int32)
        any_m = jnp.sum(mi)
        mf = m & (plsc.cumsum(mi) == 1)
        i_star = jnp.sum(jnp.where(mf, lane, 0))
        b_g = g * 16 + 15 - i_star
        hit = (found == 0) & (any_m > 0)
        b_star = jnp.where(hit, b_g, b_star)
        found = jnp.where(hit, 1, found)
        run = run + jnp.sum(tv)

    ts8 = b_star * jnp.int32(1 << 24) + sign

    def cpass(j, off):
        for u in range(4):
            o = j * 64 + u * 16
            kv = keys_ref[pl.ds(o, 16)]
            if gidx_ref is not None:
                gv = gidx_ref[pl.ds(o, 16)]
            else:
                gv = gbase + o + lane
            m = kv >= ts8
            mi = jnp.where(m, 1, 0).astype(jnp.int32)
            pos = off + plsc.cumsum(mi) - mi
            plsc.store_scatter(ck_v, [pos], kv, mask=m)
            plsc.store_scatter(cg_v, [pos], gv, mask=m)
            off = off + jnp.sum(mi)
        return off

    nc = lax.fori_loop(0, nvregs // 4, cpass, jnp.int32(0))

    pad = jnp.broadcast_to(sign, (16,))
    for u in range(4):
        ck_v[pl.ds(nc + u * 16, 16)] = pad
    nv4 = lax.shift_right_logical(nc + 63, jnp.int32(6))

    def one_bit(_, carry):
        p, bit = carry
        t = p + bit
        ts = t + sign

        def cnt(j, acc):
            for u in range(4):
                kv = ck_v[pl.ds(j * 64 + u * 16, 16)]
                acc = acc + jnp.where(kv >= ts, 1, 0).astype(jnp.int32)
            return acc

        c = jnp.sum(lax.fori_loop(0, nv4, cnt, zeros))
        p = jnp.where(c >= k, t, p)
        return p, lax.shift_right_logical(bit, jnp.int32(1))

    p0 = b_star * jnp.int32(1 << 24)
    p, _ = lax.fori_loop(0, 24, one_bit, (p0, jnp.int32(1 << 23)))
    return p + sign, nc


NW1 = 16
CHUNK1 = M_PAD // NW1
CV1 = CHUNK1 // 16
NCAND1 = NW1 * K


def _sc_fused_body(sims_hbm, mem_hbm, out_hbm,
                   chunk_v, keys_v, hist_v, tot_v,
                   ck_v, cg_v, okey_v, ogid_v,
                   k2_v, g2_v, skey_v, sgid_v, sidx_v, rows_v,
                   shk_sh, shg_sh, sem):
    wid = lax.axis_index("s")
    base = wid * CHUNK1
    pltpu.sync_copy(sims_hbm.at[pl.ds(base, CHUNK1)], chunk_v)
    t_loc, nc = _select_threshold(keys_v, CV1, hist_v, tot_v,
                                  ck_v, cg_v, K,
                                  chunk_ref=chunk_v, gbase=base)
    nv4 = lax.shift_right_logical(nc + 63, jnp.int32(6))

    def extract(j, off):
        for u in range(4):
            o = j * 64 + u * 16
            kv = ck_v[pl.ds(o, 16)]
            gv = cg_v[pl.ds(o, 16)]
            m = kv >= t_loc
            mi = jnp.where(m, 1, 0).astype(jnp.int32)
            pos = off + plsc.cumsum(mi) - mi
            plsc.store_scatter(okey_v, [pos], kv, mask=m)
            plsc.store_scatter(ogid_v, [pos], gv, mask=m)
            off = off + jnp.sum(mi)
        return off

    lax.fori_loop(0, nv4, extract, jnp.int32(0))
    pltpu.sync_copy(okey_v.at[pl.ds(0, K)], shk_sh.at[pl.ds(wid * K, K)])
    pltpu.sync_copy(ogid_v.at[pl.ds(0, K)], shg_sh.at[pl.ds(wid * K, K)])
    plsc.subcore_barrier()

    @pl.when(wid == 0)
    def _():
        pltpu.sync_copy(shk_sh, k2_v)
        pltpu.sync_copy(shg_sh, g2_v)
        t_g, nc2 = _select_threshold(k2_v, NCAND1 // 16, hist_v, tot_v,
                                     ck_v, cg_v, K, gidx_ref=g2_v)
        nv4b = lax.shift_right_logical(nc2 + 63, jnp.int32(6))

        def cnt_gt(j, acc):
            for u in range(4):
                kv = ck_v[pl.ds(j * 64 + u * 16, 16)]
                acc = acc + jnp.where(kv > t_g, 1, 0).astype(jnp.int32)
            return acc

        n_gt = jnp.sum(lax.fori_loop(0, nv4b, cnt_gt, jnp.zeros((16,), jnp.int32)))
        r = K - n_gt

        def extract2(j, carry):
            off, eq_seen = carry
            for u in range(4):
                o = j * 64 + u * 16
                kv = ck_v[pl.ds(o, 16)]
                gv = cg_v[pl.ds(o, 16)]
                m_gt = kv > t_g
                m_eq = kv == t_g
                ei = jnp.where(m_eq, 1, 0).astype(jnp.int32)
                eq_rank = eq_seen + plsc.cumsum(ei) - ei
                m = m_gt | (m_eq & (eq_rank < r))
                mi = jnp.where(m, 1, 0).astype(jnp.int32)
                pos = off + plsc.cumsum(mi) - mi
                plsc.store_scatter(skey_v, [pos], kv, mask=m)
                plsc.store_scatter(sgid_v, [pos], gv, mask=m)
                off = off + jnp.sum(mi)
                eq_seen = eq_seen + jnp.sum(ei)
            return off, eq_seen

        lax.fori_loop(0, nv4b, extract2, (jnp.int32(0), jnp.int32(0)))

        kvs = [skey_v[pl.ds(v * 16, 16)] for v in range(4)]
        gvs = [sgid_v[pl.ds(v * 16, 16)] for v in range(4)]

        def rank_step(j, ranks):
            jv = jnp.broadcast_to(j, (16,))
            kj = plsc.load_gather(skey_v, [jv])
            gj = plsc.load_gather(sgid_v, [jv])
            out = []
            for v in range(4):
                pre = (kj > kvs[v]) | ((kj == kvs[v]) & (gj < gvs[v]))
                out.append(ranks[v] + jnp.where(pre, 1, 0).astype(jnp.int32))
            return tuple(out)

        ranks = lax.fori_loop(0, K, rank_step,
                              tuple(jnp.zeros((16,), jnp.int32) for _ in range(4)))
        for v in range(4):
            plsc.store_scatter(sidx_v, [ranks[v]], gvs[v])

        pltpu.async_copy(mem_hbm.at[sidx_v], rows_v, sem).wait()
        pltpu.sync_copy(rows_v, out_hbm)


def _sc_topk_gather_fused(sims_flat, memory_tensor):
    mesh = plsc.VectorSubcoreMesh(core_axis_name="c", subcore_axis_name="s",
                                  num_cores=1)
    return pl.kernel(
        _sc_fused_body,
        out_type=jax.ShapeDtypeStruct((K, D), jnp.float32),
        mesh=mesh,
        compiler_params=pltpu.CompilerParams(needs_layout_passes=False),
        scratch_types=[
            pltpu.VMEM((CHUNK1,), jnp.float32),
            pltpu.VMEM((CHUNK1,), jnp.int32),
            pltpu.VMEM((4096,), jnp.int32),
            pltpu.VMEM((256,), jnp.int32),
            pltpu.VMEM((CHUNK1 + 64,), jnp.int32),
            pltpu.VMEM((CHUNK1 + 64,), jnp.int32),
            pltpu.VMEM((CHUNK1 + 64,), jnp.int32),
            pltpu.VMEM((CHUNK1 + 64,), jnp.int32),
            pltpu.VMEM((NCAND1,), jnp.int32),
            pltpu.VMEM((NCAND1,), jnp.int32),
            pltpu.VMEM((K,), jnp.int32),
            pltpu.VMEM((K,), jnp.int32),
            pltpu.VMEM((K,), jnp.int32),
            pltpu.VMEM((K, D), jnp.float32),
            pltpu.VMEM_SHARED((NCAND1,), jnp.int32),
            pltpu.VMEM_SHARED((NCAND1,), jnp.int32),
            pltpu.SemaphoreType.DMA,
        ],
    )(sims_flat, memory_tensor)


def _layer_norm_in(x, w, b):
    m = jnp.mean(x, axis=-1, keepdims=True)
    d = x - m
    v = jnp.mean(d * d, axis=-1, keepdims=True)
    return d * (1.0 / jnp.sqrt(v + 1e-5)) * w + b


def _tfm_body(x0_ref, wi_ref, bi_ref, wo_ref, bo_ref, w1_ref, b1_ref,
              w2_ref, b2_ref, l1w_ref, l1b_ref, l2w_ref, l2b_ref, out_ref):
    x = x0_ref[...]
    inv_sqrt = 1.0 / (DH ** 0.5)
    SH = K * H
    r6 = jax.lax.broadcasted_iota(jnp.int32, (SH, SH), 0) // K
    c6 = jax.lax.broadcasted_iota(jnp.int32, (SH, SH), 1) // K
    blk = r6 == c6
    for l in range(L):
        qkv = lax.dot_general(x, wi_ref[l], (((1,), (1,)), ((), ())),
                              preferred_element_type=jnp.float32) + bi_ref[l]
        qh = jnp.concatenate([qkv[:, h * DH:(h + 1) * DH] for h in range(H)], axis=0)
        kh = jnp.concatenate([qkv[:, D + h * DH:D + (h + 1) * DH] for h in range(H)], axis=0)
        vh = jnp.concatenate([qkv[:, 2 * D + h * DH:2 * D + (h + 1) * DH] for h in range(H)], axis=0)
        scores = jax.lax.dot_general(
            qh, kh, (((1,), (1,)), ((), ())),
            preferred_element_type=jnp.float32) * inv_sqrt
        scores = jnp.where(blk, scores, NEG)
        mx = jnp.max(scores, axis=-1, keepdims=True)
        e = jnp.exp(scores - mx)
        attn = e / jnp.sum(e, axis=-1, keepdims=True)
        oh = jnp.dot(attn, vh, preferred_element_type=jnp.float32)
        a = jnp.concatenate([oh[h * K:(h + 1) * K, :] for h in range(H)], axis=1)
        a = lax.dot_general(a, wo_ref[l], (((1,), (1,)), ((), ())),
                            preferred_element_type=jnp.float32) + bo_ref[l]
        x = _layer_norm_in(x + a, l1w_ref[l], l1b_ref[l])
        ff = jnp.maximum(lax.dot_general(x, w1_ref[l], (((1,), (1,)), ((), ())),
                                         preferred_element_type=jnp.float32) + b1_ref[l], 0.0)
        ff = lax.dot_general(ff, w2_ref[l], (((1,), (1,)), ((), ())),
                             preferred_element_type=jnp.float32) + b2_ref[l]
        x = _layer_norm_in(x + ff, l2w_ref[l], l2b_ref[l])
    out_ref[...] = x


def _transformer(x0, wi_t, bi, wo_t, bo, w1_t, b1, w2_t, b2, l1w, l1b, l2w, l2b):
    return pl.pallas_call(
        _tfm_body,
        out_shape=jax.ShapeDtypeStruct((K, D), jnp.float32),
    )(x0, wi_t, bi, wo_t, bo, w1_t, b1, w2_t, b2, l1w, l1b, l2w, l2b)


def kernel(current_context, memory_tensor, W_in, b_in, W_out, b_out,
           W1, b1, W2, b2, ln1_w, ln1_b, ln2_w, ln2_b, top_k):
    ctx_row = current_context.reshape(1, D)
    sims = _compute_sims(memory_tensor, ctx_row)
    x0 = _sc_topk_gather_fused(sims, memory_tensor)

    wi_t = W_in
    wo_t = W_out
    w1_t = W1
    w2_t = W2
    bi = b_in.reshape(L, 1, 3 * D)
    bo = b_out.reshape(L, 1, D)
    b1r = b1.reshape(L, 1, FF)
    b2r = b2.reshape(L, 1, D)
    l1w = ln1_w.reshape(L, 1, D)
    l1b = ln1_b.reshape(L, 1, D)
    l2w = ln2_w.reshape(L, 1, D)
    l2b = ln2_b.reshape(L, 1, D)
    return _transformer(x0, wi_t, bi, wo_t, bo, w1_t, b1r, w2_t, b2r,
                        l1w, l1b, l2w, l2b)
